# Initial kernel scaffold; baseline (speedup 1.0000x reference)
import jax
import jax.numpy as jnp
from jax.experimental import pallas as pl


def kernel(x, edge_index, W1, att_src1, att_dst1, b1, W2, att_src2, att_dst2, b2):
    raise NotImplementedError("write your pallas kernel here")


# speedup vs baseline: 1.0002x; 1.0002x over previous
import jax
import jax.numpy as jnp
from jax.experimental import pallas as pl
from jax.experimental.pallas import tpu as pltpu

N_NODES = 10000
HEADS1 = 8
HID = 32


def _log_softmax_body(x_ref, o_ref):
    x = x_ref[...]
    m = jnp.max(x, axis=1, keepdims=True)
    ex = jnp.exp(x - m)
    s = jnp.sum(ex, axis=1, keepdims=True)
    o_ref[...] = x - m - jnp.log(s)


def _log_softmax(x):
    n, c = x.shape
    blk = 1000
    return pl.pallas_call(
        _log_softmax_body,
        grid=(n // blk,),
        in_specs=[pl.BlockSpec((blk, c), lambda i: (i, 0))],
        out_specs=pl.BlockSpec((blk, c), lambda i: (i, 0)),
        out_shape=jax.ShapeDtypeStruct((n, c), x.dtype),
    )(x)


def _gat_conv(x, edge_index, W, att_src, att_dst, bias, heads, out_ch, concat):
    src = edge_index[0]
    dst = edge_index[1]
    n = x.shape[0]
    h = (x @ W).reshape(n, heads, out_ch)
    a_src = jnp.sum(h * att_src[None, :, :], axis=-1)
    a_dst = jnp.sum(h * att_dst[None, :, :], axis=-1)
    e = a_src[src] + a_dst[dst]
    e = jax.nn.leaky_relu(e, negative_slope=0.2)
    e_max = jax.ops.segment_max(e, dst, num_segments=n)
    e_max = jnp.where(jnp.isfinite(e_max), e_max, 0.0)
    ex = jnp.exp(e - e_max[dst])
    denom = jax.ops.segment_sum(ex, dst, num_segments=n)
    alpha = ex / (denom[dst] + 1e-16)
    msg = h[src] * alpha[:, :, None]
    out = jax.ops.segment_sum(msg, dst, num_segments=n)
    if concat:
        out = out.reshape(n, heads * out_ch)
    else:
        out = out.mean(axis=1)
    return out + bias


def kernel(x, edge_index, W1, att_src1, att_dst1, b1, W2, att_src2, att_dst2, b2):
    h1 = _gat_conv(x, edge_index, W1, att_src1, att_dst1, b1, HEADS1, HID, True)
    h1 = jax.nn.elu(h1)
    h2 = _gat_conv(h1, edge_index, W2, att_src2, att_dst2, b2, 1, 64, False)
    return _log_softmax(h2)


# speedup vs baseline: 26.0680x; 26.0628x over previous
import dataclasses
import functools

import jax
import jax.numpy as jnp
import numpy as np
from jax import lax
from jax.experimental import pallas as pl
from jax.experimental.pallas import tpu as pltpu
from jax.experimental.pallas import tpu_sc as plsc

N = 10000
E = 320000
NP = 10240
H1 = 8
C1 = 32
ROWBLK = 400

NSUB = 16
RPT = NP // NSUB

EPT1 = E // NSUB
EPT2 = E // (2 * NSUB)
CE = 1000
CM = 200

_MESH = plsc.VectorSubcoreMesh(core_axis_name="c", subcore_axis_name="s")

_SC_PARAMS = pltpu.CompilerParams()
if "needs_layout_passes" in pltpu.CompilerParams.__dataclass_fields__:
    _SC_PARAMS = dataclasses.replace(
        _SC_PARAMS, needs_layout_passes=False, use_tc_tiling_on_sc=False)


def _perm_half(s):
    p = np.arange(128)
    return (4 * s + (p % 4)) * C1 + (p // 4)


_PERM = np.concatenate([_perm_half(0), _perm_half(1)])


def _k1_body(x_ref, w_ref, h0_ref, h1_ref, as0_ref, ad0_ref, as1_ref, ad1_ref):
    y = jnp.dot(x_ref[...], w_ref[...], preferred_element_type=jnp.float32)
    h0_ref[...] = y[:, 0:128]
    h1_ref[...] = y[:, 128:256]
    as0_ref[...] = y[:, 256:272]
    ad0_ref[...] = y[:, 272:288]
    as1_ref[...] = y[:, 288:304]
    ad1_ref[...] = y[:, 304:320]


def _k1(x, wbig):
    f = pl.pallas_call(
        _k1_body,
        grid=(N // ROWBLK,),
        in_specs=[
            pl.BlockSpec((ROWBLK, 128), lambda i: (i, 0)),
            pl.BlockSpec((128, 320), lambda i: (0, 0)),
        ],
        out_specs=[
            pl.BlockSpec((ROWBLK, 128), lambda i: (i, 0)),
            pl.BlockSpec((ROWBLK, 128), lambda i: (i, 0)),
            pl.BlockSpec((ROWBLK, 16), lambda i: (i, 0)),
            pl.BlockSpec((ROWBLK, 16), lambda i: (i, 0)),
            pl.BlockSpec((ROWBLK, 16), lambda i: (i, 0)),
            pl.BlockSpec((ROWBLK, 16), lambda i: (i, 0)),
        ],
        out_shape=[
            jax.ShapeDtypeStruct((N, 128), jnp.float32),
            jax.ShapeDtypeStruct((N, 128), jnp.float32),
            jax.ShapeDtypeStruct((N, 16), jnp.float32),
            jax.ShapeDtypeStruct((N, 16), jnp.float32),
            jax.ShapeDtypeStruct((N, 16), jnp.float32),
            jax.ShapeDtypeStruct((N, 16), jnp.float32),
        ],
    )
    return f(x, wbig)


def _k5_body(o0_ref, o1_ref, b1_ref, w2_ref, v2_ref, h2_ref, a2s_ref, a2d_ref):
    h1 = jnp.concatenate([o0_ref[...], o1_ref[...]], axis=1) + b1_ref[...]
    h1 = jnp.where(h1 > 0, h1, jnp.exp(jnp.minimum(h1, 0.0)) - 1.0)
    h2_ref[...] = jnp.dot(h1, w2_ref[...], preferred_element_type=jnp.float32)
    sd = jnp.dot(h1, v2_ref[...], preferred_element_type=jnp.float32)
    a2s_ref[...] = jnp.broadcast_to(sd[:, 0:1], (sd.shape[0], 16))
    a2d_ref[...] = jnp.broadcast_to(sd[:, 1:2], (sd.shape[0], 16))


def _k5(o0, o1, b1p, w2p, v2sd):
    f = pl.pallas_call(
        _k5_body,
        grid=(N // ROWBLK,),
        in_specs=[
            pl.BlockSpec((ROWBLK, 128), lambda i: (i, 0)),
            pl.BlockSpec((ROWBLK, 128), lambda i: (i, 0)),
            pl.BlockSpec((1, 256), lambda i: (0, 0)),
            pl.BlockSpec((256, 64), lambda i: (0, 0)),
            pl.BlockSpec((256, 2), lambda i: (0, 0)),
        ],
        out_specs=[
            pl.BlockSpec((ROWBLK, 64), lambda i: (i, 0)),
            pl.BlockSpec((ROWBLK, 16), lambda i: (i, 0)),
            pl.BlockSpec((ROWBLK, 16), lambda i: (i, 0)),
        ],
        out_shape=[
            jax.ShapeDtypeStruct((N, 64), jnp.float32),
            jax.ShapeDtypeStruct((N, 16), jnp.float32),
            jax.ShapeDtypeStruct((N, 16), jnp.float32),
        ],
    )
    return f(o0, o1, b1p, w2p, v2sd)


def _k7b_body(d0_ref, d1_ref, r_ref):
    r_ref[...] = 1.0 / (d0_ref[...] + d1_ref[...] + 1e-16)


def _k7b(d0, d1):
    f = pl.pallas_call(
        _k7b_body,
        grid=(NP // RPT,),
        in_specs=[
            pl.BlockSpec((RPT, 16), lambda i: (i, 0)),
            pl.BlockSpec((RPT, 16), lambda i: (i, 0)),
        ],
        out_specs=pl.BlockSpec((RPT, 16), lambda i: (i, 0)),
        out_shape=jax.ShapeDtypeStruct((NP, 16), jnp.float32),
    )
    return f(d0, d1)


def _k9_body(p0_ref, p1_ref, b2_ref, o_ref):
    z = p0_ref[...] + p1_ref[...] + b2_ref[...]
    m = jnp.max(z, axis=1, keepdims=True)
    ex = jnp.exp(z - m)
    s = jnp.sum(ex, axis=1, keepdims=True)
    o_ref[...] = z - m - jnp.log(s)


def _k9(p0, p1, b2):
    f = pl.pallas_call(
        _k9_body,
        grid=(N // ROWBLK,),
        in_specs=[
            pl.BlockSpec((ROWBLK, 64), lambda i: (i, 0)),
            pl.BlockSpec((ROWBLK, 64), lambda i: (i, 0)),
            pl.BlockSpec((1, 64), lambda i: (0, 0)),
        ],
        out_specs=pl.BlockSpec((ROWBLK, 64), lambda i: (i, 0)),
        out_shape=jax.ShapeDtypeStruct((N, 64), jnp.float32),
    )
    return f(p0, p1, b2)


def _zero16(buf_ref, rows):
    z = jnp.zeros((16,), jnp.float32)

    def row(i, _):
        buf_ref[i] = z
        return 0

    lax.fori_loop(0, rows, row, 0)


@functools.partial(
    pl.kernel,
    out_type=[
        jax.ShapeDtypeStruct((E, 16), jnp.float32),
        jax.ShapeDtypeStruct((E, 16), jnp.float32),
        jax.ShapeDtypeStruct((NP, 16), jnp.float32),
        jax.ShapeDtypeStruct((NP, 16), jnp.float32),
    ],
    mesh=_MESH,
    compiler_params=_SC_PARAMS,
    scratch_types=[
        pltpu.VMEM((CE,), jnp.int32),
        pltpu.VMEM((CE,), jnp.int32),
        pltpu.VMEM((CE, 16), jnp.float32),
        pltpu.VMEM((CE, 16), jnp.float32),
        pltpu.VMEM((CE, 16), jnp.float32),
        pltpu.VMEM((RPT, 16), jnp.float32),
        pltpu.VMEM((16,), jnp.float32),
        pltpu.VMEM((NSUB, 16), jnp.float32),
        pltpu.VMEM_SHARED((NSUB, 16), jnp.float32),
        pltpu.VMEM_SHARED((NP, 16), jnp.float32),
        pltpu.SemaphoreType.DMA,
    ],
)
def _k2(as0, ad0, as1, ad1, src_r, dst_r, ex0, ex1, rec0, rec1,
        idxa, idxb, ga, gb, eb, zb, mrow, mxv, maxsh, den_sh, sem):
    c = lax.axis_index("c")
    t = lax.axis_index("s")
    nch = EPT1 // CE
    neg = jnp.full((16,), -3.0e38, jnp.float32)

    def half(as_ref, ad_ref, ex_ref, rec_ref):
        def chunk_a(j, m):
            base = t * EPT1 + j * CE
            pltpu.sync_copy(src_r.at[pl.ds(base, CE)], idxa)
            pltpu.async_copy(as_ref.at[idxa], ga, sem).wait()
            pltpu.sync_copy(dst_r.at[pl.ds(base, CE)], idxb)
            pltpu.async_copy(ad_ref.at[idxb], gb, sem).wait()

            def row(i, m):
                e = ga[i] + gb[i]
                e = jnp.where(e >= 0.0, e, 0.2 * e)
                eb[i] = e
                return jnp.maximum(m, e)

            m = lax.fori_loop(0, CE, row, m)
            pltpu.sync_copy(eb, ex_ref.at[pl.ds(base, CE)])
            return m

        m = lax.fori_loop(0, nch, chunk_a, neg)
        mrow[...] = m
        pltpu.sync_copy(mrow, maxsh.at[t])
        _zero16(zb, RPT)
        pltpu.sync_copy(zb, den_sh.at[pl.ds(t * RPT, RPT)])
        plsc.subcore_barrier()

        pltpu.sync_copy(maxsh, mxv)
        m = lax.fori_loop(0, NSUB, lambda i, m: jnp.maximum(m, mxv[i]), neg)
        gmax = jnp.max(m)

        def chunk_b(j, _):
            base = t * EPT1 + j * CE
            pltpu.sync_copy(ex_ref.at[pl.ds(base, CE)], eb)
            pltpu.sync_copy(dst_r.at[pl.ds(base, CE)], idxb)

            def row(i, _):
                eb[i] = jnp.exp(eb[i] - gmax)
                return 0

            lax.fori_loop(0, CE, row, 0)
            pltpu.sync_copy(eb, ex_ref.at[pl.ds(base, CE)])
            pltpu.sync_copy(eb, den_sh.at[idxb], add=True)
            return 0

        lax.fori_loop(0, nch, chunk_b, 0)
        plsc.subcore_barrier()

        r0 = t * RPT
        pltpu.sync_copy(den_sh.at[pl.ds(r0, RPT)], zb)

        def rrow(i, _):
            zb[i] = 1.0 / (zb[i] + 1e-16)
            return 0

        lax.fori_loop(0, RPT, rrow, 0)
        pltpu.sync_copy(zb, rec_ref.at[pl.ds(r0, RPT)])

    @pl.when(c == 0)
    def _():
        half(as0, ad0, ex0, rec0)

    @pl.when(c == 1)
    def _():
        half(as1, ad1, ex1, rec1)


@functools.partial(
    pl.kernel,
    out_type=[
        jax.ShapeDtypeStruct((NP, 128), jnp.float32),
        jax.ShapeDtypeStruct((NP, 128), jnp.float32),
    ],
    mesh=_MESH,
    compiler_params=_SC_PARAMS,
    scratch_types=[
        pltpu.VMEM((CM,), jnp.int32),
        pltpu.VMEM((CM,), jnp.int32),
        pltpu.VMEM((CM, 16), jnp.float32),
        pltpu.VMEM((CM, 16), jnp.float32),
        pltpu.VMEM((CM, 128), jnp.float32),
        pltpu.VMEM((128, 128), jnp.float32),
        pltpu.VMEM_SHARED((NP, 128), jnp.float32),
        pltpu.SemaphoreType.DMA,
    ],
)
def _k3(h0, h1f, ex0, ex1, rec0, rec1, src_r, dst_r, out0, out1,
        idxs, idxd, exb, recb, hb, zb, osh, sem):
    c = lax.axis_index("c")
    t = lax.axis_index("s")
    nch = EPT1 // CM
    z = jnp.zeros((16,), jnp.float32)

    def zrow(i, _):
        for v in range(8):
            zb[i, pl.ds(v * 16, 16)] = z
        return 0

    lax.fori_loop(0, 128, zrow, 0)
    for k in range(RPT // 128):
        pltpu.sync_copy(zb, osh.at[pl.ds(t * RPT + k * 128, 128)])
    plsc.subcore_barrier()

    def half(h_ref, ex_ref, rec_ref, out_ref):
        def chunk(j, _):
            base = t * EPT1 + j * CM
            pltpu.sync_copy(src_r.at[pl.ds(base, CM)], idxs)
            pltpu.async_copy(h_ref.at[idxs], hb, sem).wait()
            pltpu.sync_copy(dst_r.at[pl.ds(base, CM)], idxd)
            pltpu.async_copy(rec_ref.at[idxd], recb, sem).wait()
            pltpu.sync_copy(ex_ref.at[pl.ds(base, CM)], exb)

            def row(i, _):
                alpha = exb[i] * recb[i]
                for v in range(8):
                    sl = pl.ds(v * 16, 16)
                    hb[i, sl] = hb[i, sl] * alpha
                return 0

            lax.fori_loop(0, CM, row, 0)
            pltpu.sync_copy(hb, osh.at[idxd], add=True)
            return 0

        lax.fori_loop(0, nch, chunk, 0)
        plsc.subcore_barrier()
        r0 = t * RPT
        pltpu.sync_copy(osh.at[pl.ds(r0, RPT)], out_ref.at[pl.ds(r0, RPT)])

    @pl.when(c == 0)
    def _():
        half(h0, ex0, rec0, out0)

    @pl.when(c == 1)
    def _():
        half(h1f, ex1, rec1, out1)


@functools.partial(
    pl.kernel,
    out_type=[
        jax.ShapeDtypeStruct((E, 16), jnp.float32),
        jax.ShapeDtypeStruct((2, 16), jnp.float32),
    ],
    mesh=_MESH,
    compiler_params=_SC_PARAMS,
    scratch_types=[
        pltpu.VMEM((CE,), jnp.int32),
        pltpu.VMEM((CE,), jnp.int32),
        pltpu.VMEM((CE, 16), jnp.float32),
        pltpu.VMEM((CE, 16), jnp.float32),
        pltpu.VMEM((CE, 16), jnp.float32),
        pltpu.VMEM((16,), jnp.float32),
        pltpu.VMEM((NSUB, 16), jnp.float32),
        pltpu.VMEM_SHARED((NSUB, 16), jnp.float32),
        pltpu.SemaphoreType.DMA,
    ],
)
def _k6(a2s, a2d, src_r, dst_r, e2s, mx2,
        idxa, idxb, ga, gb, eb, mrow, mxv, maxsh, sem):
    c = lax.axis_index("c")
    t = lax.axis_index("s")
    nch = EPT2 // CE
    neg = jnp.full((16,), -3.0e38, jnp.float32)

    def chunk_a(j, m):
        base = c * (E // 2) + t * EPT2 + j * CE
        pltpu.sync_copy(src_r.at[pl.ds(base, CE)], idxa)
        pltpu.async_copy(a2s.at[idxa], ga, sem).wait()
        pltpu.sync_copy(dst_r.at[pl.ds(base, CE)], idxb)
        pltpu.async_copy(a2d.at[idxb], gb, sem).wait()

        def row(i, m):
            e = ga[i] + gb[i]
            e = jnp.where(e >= 0.0, e, 0.2 * e)
            eb[i] = e
            return jnp.maximum(m, e)

        m = lax.fori_loop(0, CE, row, m)
        pltpu.sync_copy(eb, e2s.at[pl.ds(base, CE)])
        return m

    m = lax.fori_loop(0, nch, chunk_a, neg)
    mrow[...] = m
    pltpu.sync_copy(mrow, maxsh.at[t])
    plsc.subcore_barrier()
    pltpu.sync_copy(maxsh, mxv)
    m = lax.fori_loop(0, NSUB, lambda i, m: jnp.maximum(m, mxv[i]), neg)
    mrow[...] = m

    @pl.when(jnp.logical_and(t == 0, c == 0))
    def _():
        pltpu.sync_copy(mrow, mx2.at[0])

    @pl.when(jnp.logical_and(t == 0, c == 1))
    def _():
        pltpu.sync_copy(mrow, mx2.at[1])


@functools.partial(
    pl.kernel,
    out_type=[
        jax.ShapeDtypeStruct((E, 16), jnp.float32),
        jax.ShapeDtypeStruct((NP, 16), jnp.float32),
        jax.ShapeDtypeStruct((NP, 16), jnp.float32),
    ],
    mesh=_MESH,
    compiler_params=_SC_PARAMS,
    scratch_types=[
        pltpu.VMEM((CE,), jnp.int32),
        pltpu.VMEM((CE, 16), jnp.float32),
        pltpu.VMEM((RPT, 16), jnp.float32),
        pltpu.VMEM((2, 16), jnp.float32),
        pltpu.VMEM_SHARED((NP, 16), jnp.float32),
        pltpu.SemaphoreType.DMA,
    ],
)
def _k7(e2s, mx2, dst_r, ex2s, d20, d21, idxb, eb, zb, mxv2, den_sh, sem):
    c = lax.axis_index("c")
    t = lax.axis_index("s")
    nch = EPT2 // CE

    pltpu.sync_copy(mx2, mxv2)
    gmax = jnp.max(jnp.maximum(mxv2[0], mxv2[1]))

    _zero16(zb, RPT)
    pltpu.sync_copy(zb, den_sh.at[pl.ds(t * RPT, RPT)])
    plsc.subcore_barrier()

    def chunk(j, _):
        base = c * (E // 2) + t * EPT2 + j * CE
        pltpu.sync_copy(e2s.at[pl.ds(base, CE)], eb)
        pltpu.sync_copy(dst_r.at[pl.ds(base, CE)], idxb)

        def row(i, _):
            eb[i] = jnp.exp(eb[i] - gmax)
            return 0

        lax.fori_loop(0, CE, row, 0)
        pltpu.sync_copy(eb, ex2s.at[pl.ds(base, CE)])
        pltpu.sync_copy(eb, den_sh.at[idxb], add=True)
        return 0

    lax.fori_loop(0, nch, chunk, 0)
    plsc.subcore_barrier()

    r0 = t * RPT

    @pl.when(c == 0)
    def _():
        pltpu.sync_copy(den_sh.at[pl.ds(r0, RPT)], d20.at[pl.ds(r0, RPT)])

    @pl.when(c == 1)
    def _():
        pltpu.sync_copy(den_sh.at[pl.ds(r0, RPT)], d21.at[pl.ds(r0, RPT)])


@functools.partial(
    pl.kernel,
    out_type=[
        jax.ShapeDtypeStruct((NP, 64), jnp.float32),
        jax.ShapeDtypeStruct((NP, 64), jnp.float32),
    ],
    mesh=_MESH,
    compiler_params=_SC_PARAMS,
    scratch_types=[
        pltpu.VMEM((CM,), jnp.int32),
        pltpu.VMEM((CM,), jnp.int32),
        pltpu.VMEM((CM, 16), jnp.float32),
        pltpu.VMEM((CM, 16), jnp.float32),
        pltpu.VMEM((CM, 64), jnp.float32),
        pltpu.VMEM((128, 64), jnp.float32),
        pltpu.VMEM_SHARED((NP, 64), jnp.float32),
        pltpu.SemaphoreType.DMA,
    ],
)
def _k8(h2, ex2s, rec2, src_r, dst_r, o2p0, o2p1,
        idxs, idxd, exb, recb, hb, zb, osh, sem):
    c = lax.axis_index("c")
    t = lax.axis_index("s")
    nch = EPT2 // CM
    z = jnp.zeros((16,), jnp.float32)

    def zrow(i, _):
        for v in range(4):
            zb[i, pl.ds(v * 16, 16)] = z
        return 0

    lax.fori_loop(0, 128, zrow, 0)
    for k in range(RPT // 128):
        pltpu.sync_copy(zb, osh.at[pl.ds(t * RPT + k * 128, 128)])
    plsc.subcore_barrier()

    def chunk(j, _):
        base = c * (E // 2) + t * EPT2 + j * CM
        pltpu.sync_copy(src_r.at[pl.ds(base, CM)], idxs)
        pltpu.async_copy(h2.at[idxs], hb, sem).wait()
        pltpu.sync_copy(dst_r.at[pl.ds(base, CM)], idxd)
        pltpu.async_copy(rec2.at[idxd], recb, sem).wait()
        pltpu.sync_copy(ex2s.at[pl.ds(base, CM)], exb)

        def row(i, _):
            alpha = exb[i] * recb[i]
            for v in range(4):
                sl = pl.ds(v * 16, 16)
                hb[i, sl] = hb[i, sl] * alpha
            return 0

        lax.fori_loop(0, CM, row, 0)
        pltpu.sync_copy(hb, osh.at[idxd], add=True)
        return 0

    lax.fori_loop(0, nch, chunk, 0)
    plsc.subcore_barrier()

    r0 = t * RPT

    @pl.when(c == 0)
    def _():
        pltpu.sync_copy(osh.at[pl.ds(r0, RPT)], o2p0.at[pl.ds(r0, RPT)])

    @pl.when(c == 1)
    def _():
        pltpu.sync_copy(osh.at[pl.ds(r0, RPT)], o2p1.at[pl.ds(r0, RPT)])


def kernel(x, edge_index, W1, att_src1, att_dst1, b1, W2, att_src2, att_dst2, b2):
    src = edge_index[0]
    dst = edge_index[1]

    perm = jnp.asarray(_PERM)
    w1p = W1[:, perm]
    v_src = jnp.sum(W1.reshape(128, H1, C1) * att_src1[None], axis=-1)
    v_dst = jnp.sum(W1.reshape(128, H1, C1) * att_dst1[None], axis=-1)
    rep = jnp.asarray(np.tile(np.arange(4), 4))
    wbig = jnp.concatenate(
        [w1p,
         v_src[:, 0 + rep], v_dst[:, 0 + rep],
         v_src[:, 4 + rep], v_dst[:, 4 + rep]], axis=1)
    b1p = b1[perm][None, :]
    w2p = W2[perm, :]
    v2sd = jnp.stack([w2p @ att_src2[0], w2p @ att_dst2[0]], axis=1)

    h0, h1f, as0, ad0, as1, ad1 = _k1(x, wbig)
    ex0, ex1, rec0, rec1 = _k2(as0, ad0, as1, ad1, src, dst)
    out0, out1 = _k3(h0, h1f, ex0, ex1, rec0, rec1, src, dst)

    h2, a2s_t, a2d_t = _k5(out0, out1, b1p, w2p, v2sd)
    e2s, mx2 = _k6(a2s_t, a2d_t, src, dst)
    ex2s, d20, d21 = _k7(e2s, mx2, dst)
    rec2 = _k7b(d20, d21)
    o2p0, o2p1 = _k8(h2, ex2s, rec2, src, dst)
    return _k9(o2p0, o2p1, b2[None, :])


# speedup vs baseline: 33.4713x; 1.2840x over previous
import dataclasses
import functools

import jax
import jax.numpy as jnp
import numpy as np
from jax import lax
from jax.experimental import pallas as pl
from jax.experimental.pallas import tpu as pltpu
from jax.experimental.pallas import tpu_sc as plsc

N = 10000
E = 320000
NP = 10240
H1 = 8
C1 = 32
ROWBLK = 400

NSUB = 16
RPT = NP // NSUB

EPT1 = E // NSUB
EPT2 = E // (2 * NSUB)
CE = 1000
CM = 200

_MESH = plsc.VectorSubcoreMesh(core_axis_name="c", subcore_axis_name="s")

_SC_PARAMS = pltpu.CompilerParams()
if "needs_layout_passes" in pltpu.CompilerParams.__dataclass_fields__:
    _SC_PARAMS = dataclasses.replace(
        _SC_PARAMS, needs_layout_passes=False, use_tc_tiling_on_sc=False)


def _perm_half(s):
    p = np.arange(128)
    return (4 * s + (p % 4)) * C1 + (p // 4)


_PERM = np.concatenate([_perm_half(0), _perm_half(1)])


def _k1_body(x_ref, w_ref, h0_ref, h1_ref, as0_ref, ad0_ref, as1_ref, ad1_ref,
             mx_ref):
    i = pl.program_id(0)
    y = jnp.dot(x_ref[...], w_ref[...], preferred_element_type=jnp.float32)
    h0_ref[...] = y[:, 0:128]
    h1_ref[...] = y[:, 128:256]
    as0_ref[...] = y[:, 256:272]
    ad0_ref[...] = y[:, 272:288]
    as1_ref[...] = y[:, 288:304]
    ad1_ref[...] = y[:, 304:320]
    blkmax = jnp.concatenate(
        [jnp.max(y[:, 256 + 16 * k:272 + 16 * k], axis=0)[None] for k in range(4)]
        + [jnp.full((4, 16), -3.0e38, jnp.float32)], axis=0)

    @pl.when(i == 0)
    def _():
        mx_ref[...] = jnp.full((8, 16), -3.0e38, jnp.float32)

    mx_ref[...] = jnp.maximum(mx_ref[...], blkmax)


def _k1(x, wbig):
    f = pl.pallas_call(
        _k1_body,
        grid=(N // ROWBLK,),
        in_specs=[
            pl.BlockSpec((ROWBLK, 128), lambda i: (i, 0)),
            pl.BlockSpec((128, 320), lambda i: (0, 0)),
        ],
        out_specs=[
            pl.BlockSpec((ROWBLK, 128), lambda i: (i, 0)),
            pl.BlockSpec((ROWBLK, 128), lambda i: (i, 0)),
            pl.BlockSpec((ROWBLK, 16), lambda i: (i, 0)),
            pl.BlockSpec((ROWBLK, 16), lambda i: (i, 0)),
            pl.BlockSpec((ROWBLK, 16), lambda i: (i, 0)),
            pl.BlockSpec((ROWBLK, 16), lambda i: (i, 0)),
            pl.BlockSpec((8, 16), lambda i: (0, 0)),
        ],
        out_shape=[
            jax.ShapeDtypeStruct((N, 128), jnp.float32),
            jax.ShapeDtypeStruct((N, 128), jnp.float32),
            jax.ShapeDtypeStruct((N, 16), jnp.float32),
            jax.ShapeDtypeStruct((N, 16), jnp.float32),
            jax.ShapeDtypeStruct((N, 16), jnp.float32),
            jax.ShapeDtypeStruct((N, 16), jnp.float32),
            jax.ShapeDtypeStruct((8, 16), jnp.float32),
        ],
    )
    return f(x, wbig)


def _k5_body(o0_ref, o1_ref, b1_ref, w2_ref, v2_ref, h2_ref, a2s_ref, a2d_ref,
             mx_ref):
    i = pl.program_id(0)
    h1 = jnp.concatenate([o0_ref[...], o1_ref[...]], axis=1) + b1_ref[...]
    h1 = jnp.where(h1 > 0, h1, jnp.exp(jnp.minimum(h1, 0.0)) - 1.0)
    h2_ref[...] = jnp.dot(h1, w2_ref[...], preferred_element_type=jnp.float32)
    sd = jnp.dot(h1, v2_ref[...], preferred_element_type=jnp.float32)
    a2s = jnp.broadcast_to(sd[:, 0:1], (sd.shape[0], 16))
    a2d = jnp.broadcast_to(sd[:, 1:2], (sd.shape[0], 16))
    a2s_ref[...] = a2s
    a2d_ref[...] = a2d
    blkmax = jnp.concatenate(
        [jnp.max(a2s, axis=0)[None], jnp.max(a2d, axis=0)[None],
         jnp.full((6, 16), -3.0e38, jnp.float32)], axis=0)

    @pl.when(i == 0)
    def _():
        mx_ref[...] = jnp.full((8, 16), -3.0e38, jnp.float32)

    mx_ref[...] = jnp.maximum(mx_ref[...], blkmax)


def _k5(o0, o1, b1p, w2p, v2sd):
    f = pl.pallas_call(
        _k5_body,
        grid=(N // ROWBLK,),
        in_specs=[
            pl.BlockSpec((ROWBLK, 128), lambda i: (i, 0)),
            pl.BlockSpec((ROWBLK, 128), lambda i: (i, 0)),
            pl.BlockSpec((1, 256), lambda i: (0, 0)),
            pl.BlockSpec((256, 64), lambda i: (0, 0)),
            pl.BlockSpec((256, 2), lambda i: (0, 0)),
        ],
        out_specs=[
            pl.BlockSpec((ROWBLK, 64), lambda i: (i, 0)),
            pl.BlockSpec((ROWBLK, 16), lambda i: (i, 0)),
            pl.BlockSpec((ROWBLK, 16), lambda i: (i, 0)),
            pl.BlockSpec((8, 16), lambda i: (0, 0)),
        ],
        out_shape=[
            jax.ShapeDtypeStruct((N, 64), jnp.float32),
            jax.ShapeDtypeStruct((N, 16), jnp.float32),
            jax.ShapeDtypeStruct((N, 16), jnp.float32),
            jax.ShapeDtypeStruct((8, 16), jnp.float32),
        ],
    )
    return f(o0, o1, b1p, w2p, v2sd)


def _k7b_body(d0_ref, d1_ref, r_ref):
    r_ref[...] = 1.0 / (d0_ref[...] + d1_ref[...] + 1e-16)


def _k7b(d0, d1):
    f = pl.pallas_call(
        _k7b_body,
        grid=(NP // RPT,),
        in_specs=[
            pl.BlockSpec((RPT, 16), lambda i: (i, 0)),
            pl.BlockSpec((RPT, 16), lambda i: (i, 0)),
        ],
        out_specs=pl.BlockSpec((RPT, 16), lambda i: (i, 0)),
        out_shape=jax.ShapeDtypeStruct((NP, 16), jnp.float32),
    )
    return f(d0, d1)


def _k9_body(p0_ref, p1_ref, b2_ref, o_ref):
    z = p0_ref[...] + p1_ref[...] + b2_ref[...]
    m = jnp.max(z, axis=1, keepdims=True)
    ex = jnp.exp(z - m)
    s = jnp.sum(ex, axis=1, keepdims=True)
    o_ref[...] = z - m - jnp.log(s)


def _k9(p0, p1, b2):
    f = pl.pallas_call(
        _k9_body,
        grid=(N // ROWBLK,),
        in_specs=[
            pl.BlockSpec((ROWBLK, 64), lambda i: (i, 0)),
            pl.BlockSpec((ROWBLK, 64), lambda i: (i, 0)),
            pl.BlockSpec((1, 64), lambda i: (0, 0)),
        ],
        out_specs=pl.BlockSpec((ROWBLK, 64), lambda i: (i, 0)),
        out_shape=jax.ShapeDtypeStruct((N, 64), jnp.float32),
    )
    return f(p0, p1, b2)


def _zero16(buf_ref, rows):
    z = jnp.zeros((16,), jnp.float32)

    def row(i, _):
        buf_ref[i] = z
        return 0

    lax.fori_loop(0, rows, row, 0)


@functools.partial(
    pl.kernel,
    out_type=[
        jax.ShapeDtypeStruct((E, 16), jnp.float32),
        jax.ShapeDtypeStruct((E, 16), jnp.float32),
        jax.ShapeDtypeStruct((NP, 16), jnp.float32),
        jax.ShapeDtypeStruct((NP, 16), jnp.float32),
    ],
    mesh=_MESH,
    compiler_params=_SC_PARAMS,
    scratch_types=[
        pltpu.VMEM((CE,), jnp.int32),
        pltpu.VMEM((CE,), jnp.int32),
        pltpu.VMEM((CE, 16), jnp.float32),
        pltpu.VMEM((CE, 16), jnp.float32),
        pltpu.VMEM((CE, 16), jnp.float32),
        pltpu.VMEM((RPT, 16), jnp.float32),
        pltpu.VMEM((8, 16), jnp.float32),
        pltpu.VMEM_SHARED((NP, 16), jnp.float32),
        pltpu.SemaphoreType.DMA,
    ],
)
def _k2(as0, ad0, as1, ad1, mx1, src_r, dst_r, ex0, ex1, rec0, rec1,
        idxa, idxb, ga, gb, eb, zb, mxb, den_sh, sem):
    c = lax.axis_index("c")
    t = lax.axis_index("s")
    nch = EPT1 // CE

    pltpu.sync_copy(mx1, mxb)
    _zero16(zb, RPT)
    pltpu.sync_copy(zb, den_sh.at[pl.ds(t * RPT, RPT)])
    plsc.subcore_barrier()

    def half(s, as_ref, ad_ref, ex_ref, rec_ref):
        shift = jnp.maximum(mxb[2 * s] + mxb[2 * s + 1], 0.0)

        def chunk(j, _):
            base = t * EPT1 + j * CE
            pltpu.sync_copy(src_r.at[pl.ds(base, CE)], idxa)
            pltpu.async_copy(as_ref.at[idxa], ga, sem).wait()
            pltpu.sync_copy(dst_r.at[pl.ds(base, CE)], idxb)
            pltpu.async_copy(ad_ref.at[idxb], gb, sem).wait()

            def row(i, _):
                e = ga[i] + gb[i]
                e = jnp.where(e >= 0.0, e, 0.2 * e)
                eb[i] = jnp.exp(e - shift)
                return 0

            lax.fori_loop(0, CE, row, 0)
            pltpu.sync_copy(eb, ex_ref.at[pl.ds(base, CE)])
            pltpu.sync_copy(eb, den_sh.at[idxb], add=True)
            return 0

        lax.fori_loop(0, nch, chunk, 0)
        plsc.subcore_barrier()

        r0 = t * RPT
        pltpu.sync_copy(den_sh.at[pl.ds(r0, RPT)], zb)

        def rrow(i, _):
            zb[i] = 1.0 / (zb[i] + 1e-16)
            return 0

        lax.fori_loop(0, RPT, rrow, 0)
        pltpu.sync_copy(zb, rec_ref.at[pl.ds(r0, RPT)])

    @pl.when(c == 0)
    def _():
        half(0, as0, ad0, ex0, rec0)

    @pl.when(c == 1)
    def _():
        half(1, as1, ad1, ex1, rec1)


@functools.partial(
    pl.kernel,
    out_type=[
        jax.ShapeDtypeStruct((NP, 128), jnp.float32),
        jax.ShapeDtypeStruct((NP, 128), jnp.float32),
    ],
    mesh=_MESH,
    compiler_params=_SC_PARAMS,
    scratch_types=[
        pltpu.VMEM((CM,), jnp.int32),
        pltpu.VMEM((CM,), jnp.int32),
        pltpu.VMEM((CM, 16), jnp.float32),
        pltpu.VMEM((CM, 16), jnp.float32),
        pltpu.VMEM((CM, 128), jnp.float32),
        pltpu.VMEM((128, 128), jnp.float32),
        pltpu.VMEM_SHARED((NP, 128), jnp.float32),
        pltpu.SemaphoreType.DMA,
    ],
)
def _k3(h0, h1f, ex0, ex1, rec0, rec1, src_r, dst_r, out0, out1,
        idxs, idxd, exb, recb, hb, zb, osh, sem):
    c = lax.axis_index("c")
    t = lax.axis_index("s")
    nch = EPT1 // CM
    z = jnp.zeros((16,), jnp.float32)

    def zrow(i, _):
        for v in range(8):
            zb[i, pl.ds(v * 16, 16)] = z
        return 0

    lax.fori_loop(0, 128, zrow, 0)
    for k in range(RPT // 128):
        pltpu.sync_copy(zb, osh.at[pl.ds(t * RPT + k * 128, 128)])
    plsc.subcore_barrier()

    def half(h_ref, ex_ref, rec_ref, out_ref):
        def chunk(j, _):
            base = t * EPT1 + j * CM
            pltpu.sync_copy(src_r.at[pl.ds(base, CM)], idxs)
            pltpu.async_copy(h_ref.at[idxs], hb, sem).wait()
            pltpu.sync_copy(dst_r.at[pl.ds(base, CM)], idxd)
            pltpu.async_copy(rec_ref.at[idxd], recb, sem).wait()
            pltpu.sync_copy(ex_ref.at[pl.ds(base, CM)], exb)

            def row(i, _):
                alpha = exb[i] * recb[i]
                for v in range(8):
                    sl = pl.ds(v * 16, 16)
                    hb[i, sl] = hb[i, sl] * alpha
                return 0

            lax.fori_loop(0, CM, row, 0)
            pltpu.sync_copy(hb, osh.at[idxd], add=True)
            return 0

        lax.fori_loop(0, nch, chunk, 0)
        plsc.subcore_barrier()
        r0 = t * RPT
        pltpu.sync_copy(osh.at[pl.ds(r0, RPT)], out_ref.at[pl.ds(r0, RPT)])

    @pl.when(c == 0)
    def _():
        half(h0, ex0, rec0, out0)

    @pl.when(c == 1)
    def _():
        half(h1f, ex1, rec1, out1)


@functools.partial(
    pl.kernel,
    out_type=[
        jax.ShapeDtypeStruct((E, 16), jnp.float32),
        jax.ShapeDtypeStruct((NP, 16), jnp.float32),
        jax.ShapeDtypeStruct((NP, 16), jnp.float32),
    ],
    mesh=_MESH,
    compiler_params=_SC_PARAMS,
    scratch_types=[
        pltpu.VMEM((CE,), jnp.int32),
        pltpu.VMEM((CE,), jnp.int32),
        pltpu.VMEM((CE, 16), jnp.float32),
        pltpu.VMEM((CE, 16), jnp.float32),
        pltpu.VMEM((CE, 16), jnp.float32),
        pltpu.VMEM((RPT, 16), jnp.float32),
        pltpu.VMEM((8, 16), jnp.float32),
        pltpu.VMEM_SHARED((NP, 16), jnp.float32),
        pltpu.SemaphoreType.DMA,
    ],
)
def _k7(a2s, a2d, mx2, src_r, dst_r, ex2s, d20, d21,
        idxa, idxb, ga, gb, eb, zb, mxb, den_sh, sem):
    c = lax.axis_index("c")
    t = lax.axis_index("s")
    nch = EPT2 // CE

    pltpu.sync_copy(mx2, mxb)
    shift = jnp.maximum(mxb[0] + mxb[1], 0.0)

    _zero16(zb, RPT)
    pltpu.sync_copy(zb, den_sh.at[pl.ds(t * RPT, RPT)])
    plsc.subcore_barrier()

    def chunk(j, _):
        base = c * (E // 2) + t * EPT2 + j * CE
        pltpu.sync_copy(src_r.at[pl.ds(base, CE)], idxa)
        pltpu.async_copy(a2s.at[idxa], ga, sem).wait()
        pltpu.sync_copy(dst_r.at[pl.ds(base, CE)], idxb)
        pltpu.async_copy(a2d.at[idxb], gb, sem).wait()

        def row(i, _):
            e = ga[i] + gb[i]
            e = jnp.where(e >= 0.0, e, 0.2 * e)
            eb[i] = jnp.exp(e - shift)
            return 0

        lax.fori_loop(0, CE, row, 0)
        pltpu.sync_copy(eb, ex2s.at[pl.ds(base, CE)])
        pltpu.sync_copy(eb, den_sh.at[idxb], add=True)
        return 0

    lax.fori_loop(0, nch, chunk, 0)
    plsc.subcore_barrier()

    r0 = t * RPT

    @pl.when(c == 0)
    def _():
        pltpu.sync_copy(den_sh.at[pl.ds(r0, RPT)], d20.at[pl.ds(r0, RPT)])

    @pl.when(c == 1)
    def _():
        pltpu.sync_copy(den_sh.at[pl.ds(r0, RPT)], d21.at[pl.ds(r0, RPT)])


@functools.partial(
    pl.kernel,
    out_type=[
        jax.ShapeDtypeStruct((NP, 64), jnp.float32),
        jax.ShapeDtypeStruct((NP, 64), jnp.float32),
    ],
    mesh=_MESH,
    compiler_params=_SC_PARAMS,
    scratch_types=[
        pltpu.VMEM((CM,), jnp.int32),
        pltpu.VMEM((CM,), jnp.int32),
        pltpu.VMEM((CM, 16), jnp.float32),
        pltpu.VMEM((CM, 16), jnp.float32),
        pltpu.VMEM((CM, 64), jnp.float32),
        pltpu.VMEM((128, 64), jnp.float32),
        pltpu.VMEM_SHARED((NP, 64), jnp.float32),
        pltpu.SemaphoreType.DMA,
    ],
)
def _k8(h2, ex2s, rec2, src_r, dst_r, o2p0, o2p1,
        idxs, idxd, exb, recb, hb, zb, osh, sem):
    c = lax.axis_index("c")
    t = lax.axis_index("s")
    nch = EPT2 // CM
    z = jnp.zeros((16,), jnp.float32)

    def zrow(i, _):
        for v in range(4):
            zb[i, pl.ds(v * 16, 16)] = z
        return 0

    lax.fori_loop(0, 128, zrow, 0)
    for k in range(RPT // 128):
        pltpu.sync_copy(zb, osh.at[pl.ds(t * RPT + k * 128, 128)])
    plsc.subcore_barrier()

    def chunk(j, _):
        base = c * (E // 2) + t * EPT2 + j * CM
        pltpu.sync_copy(src_r.at[pl.ds(base, CM)], idxs)
        pltpu.async_copy(h2.at[idxs], hb, sem).wait()
        pltpu.sync_copy(dst_r.at[pl.ds(base, CM)], idxd)
        pltpu.async_copy(rec2.at[idxd], recb, sem).wait()
        pltpu.sync_copy(ex2s.at[pl.ds(base, CM)], exb)

        def row(i, _):
            alpha = exb[i] * recb[i]
            for v in range(4):
                sl = pl.ds(v * 16, 16)
                hb[i, sl] = hb[i, sl] * alpha
            return 0

        lax.fori_loop(0, CM, row, 0)
        pltpu.sync_copy(hb, osh.at[idxd], add=True)
        return 0

    lax.fori_loop(0, nch, chunk, 0)
    plsc.subcore_barrier()

    r0 = t * RPT

    @pl.when(c == 0)
    def _():
        pltpu.sync_copy(osh.at[pl.ds(r0, RPT)], o2p0.at[pl.ds(r0, RPT)])

    @pl.when(c == 1)
    def _():
        pltpu.sync_copy(osh.at[pl.ds(r0, RPT)], o2p1.at[pl.ds(r0, RPT)])


def kernel(x, edge_index, W1, att_src1, att_dst1, b1, W2, att_src2, att_dst2, b2):
    src = edge_index[0]
    dst = edge_index[1]

    perm = jnp.asarray(_PERM)
    w1p = W1[:, perm]
    v_src = jnp.sum(W1.reshape(128, H1, C1) * att_src1[None], axis=-1)
    v_dst = jnp.sum(W1.reshape(128, H1, C1) * att_dst1[None], axis=-1)
    rep = jnp.asarray(np.tile(np.arange(4), 4))
    wbig = jnp.concatenate(
        [w1p,
         v_src[:, 0 + rep], v_dst[:, 0 + rep],
         v_src[:, 4 + rep], v_dst[:, 4 + rep]], axis=1)
    b1p = b1[perm][None, :]
    w2p = W2[perm, :]
    v2sd = jnp.stack([w2p @ att_src2[0], w2p @ att_dst2[0]], axis=1)

    h0, h1f, as0, ad0, as1, ad1, mx1 = _k1(x, wbig)
    ex0, ex1, rec0, rec1 = _k2(as0, ad0, as1, ad1, mx1, src, dst)
    out0, out1 = _k3(h0, h1f, ex0, ex1, rec0, rec1, src, dst)

    h2, a2s_t, a2d_t, mx2 = _k5(out0, out1, b1p, w2p, v2sd)
    ex2s, d20, d21 = _k7(a2s_t, a2d_t, mx2, src, dst)
    rec2 = _k7b(d20, d21)
    o2p0, o2p1 = _k8(h2, ex2s, rec2, src, dst)
    return _k9(o2p0, o2p1, b2[None, :])


# speedup vs baseline: 42.3126x; 1.2641x over previous
import dataclasses
import functools

import jax
import jax.numpy as jnp
import numpy as np
from jax import lax
from jax.experimental import pallas as pl
from jax.experimental.pallas import tpu as pltpu
from jax.experimental.pallas import tpu_sc as plsc

N = 10000
E = 320000
NP = 10240
H1 = 8
C1 = 32
ROWBLK = 400

NSUB = 16
RPT = NP // NSUB

EPT1 = E // NSUB
EPT2 = E // (2 * NSUB)
CE = 1000
CM = 200
CM3 = 80

_MESH = plsc.VectorSubcoreMesh(core_axis_name="c", subcore_axis_name="s")

_SC_PARAMS = pltpu.CompilerParams()
if "needs_layout_passes" in pltpu.CompilerParams.__dataclass_fields__:
    _SC_PARAMS = dataclasses.replace(
        _SC_PARAMS, needs_layout_passes=False, use_tc_tiling_on_sc=False)


def _perm_half(s):
    p = np.arange(128)
    return (4 * s + (p % 4)) * C1 + (p // 4)


_PERM = np.concatenate([_perm_half(0), _perm_half(1)])


def _k1_body(x_ref, w_ref, h0_ref, h1_ref, as0_ref, ad0_ref, as1_ref, ad1_ref,
             mx_ref):
    i = pl.program_id(0)
    y = jnp.dot(x_ref[...], w_ref[...], preferred_element_type=jnp.float32)
    h0_ref[...] = y[:, 0:128]
    h1_ref[...] = y[:, 128:256]
    as0_ref[...] = y[:, 256:272]
    ad0_ref[...] = y[:, 272:288]
    as1_ref[...] = y[:, 288:304]
    ad1_ref[...] = y[:, 304:320]
    blkmax = jnp.concatenate(
        [jnp.max(y[:, 256 + 16 * k:272 + 16 * k], axis=0)[None] for k in range(4)]
        + [jnp.full((4, 16), -3.0e38, jnp.float32)], axis=0)

    @pl.when(i == 0)
    def _():
        mx_ref[...] = jnp.full((8, 16), -3.0e38, jnp.float32)

    mx_ref[...] = jnp.maximum(mx_ref[...], blkmax)


def _k1(x, wbig):
    f = pl.pallas_call(
        _k1_body,
        grid=(N // ROWBLK,),
        in_specs=[
            pl.BlockSpec((ROWBLK, 128), lambda i: (i, 0)),
            pl.BlockSpec((128, 320), lambda i: (0, 0)),
        ],
        out_specs=[
            pl.BlockSpec((ROWBLK, 128), lambda i: (i, 0)),
            pl.BlockSpec((ROWBLK, 128), lambda i: (i, 0)),
            pl.BlockSpec((ROWBLK, 16), lambda i: (i, 0)),
            pl.BlockSpec((ROWBLK, 16), lambda i: (i, 0)),
            pl.BlockSpec((ROWBLK, 16), lambda i: (i, 0)),
            pl.BlockSpec((ROWBLK, 16), lambda i: (i, 0)),
            pl.BlockSpec((8, 16), lambda i: (0, 0)),
        ],
        out_shape=[
            jax.ShapeDtypeStruct((N, 128), jnp.float32),
            jax.ShapeDtypeStruct((N, 128), jnp.float32),
            jax.ShapeDtypeStruct((N, 16), jnp.float32),
            jax.ShapeDtypeStruct((N, 16), jnp.float32),
            jax.ShapeDtypeStruct((N, 16), jnp.float32),
            jax.ShapeDtypeStruct((N, 16), jnp.float32),
            jax.ShapeDtypeStruct((8, 16), jnp.float32),
        ],
    )
    return f(x, wbig)


def _k5_body(o0_ref, o1_ref, b1_ref, w2_ref, v2_ref, h2_ref, a2s_ref, a2d_ref,
             mx_ref):
    i = pl.program_id(0)
    h1 = jnp.concatenate([o0_ref[...], o1_ref[...]], axis=1) + b1_ref[...]
    h1 = jnp.where(h1 > 0, h1, jnp.exp(jnp.minimum(h1, 0.0)) - 1.0)
    h2_ref[...] = jnp.dot(h1, w2_ref[...], preferred_element_type=jnp.float32)
    sd = jnp.dot(h1, v2_ref[...], preferred_element_type=jnp.float32)
    a2s = jnp.broadcast_to(sd[:, 0:1], (sd.shape[0], 16))
    a2d = jnp.broadcast_to(sd[:, 1:2], (sd.shape[0], 16))
    a2s_ref[...] = a2s
    a2d_ref[...] = a2d
    blkmax = jnp.concatenate(
        [jnp.max(a2s, axis=0)[None], jnp.max(a2d, axis=0)[None],
         jnp.full((6, 16), -3.0e38, jnp.float32)], axis=0)

    @pl.when(i == 0)
    def _():
        mx_ref[...] = jnp.full((8, 16), -3.0e38, jnp.float32)

    mx_ref[...] = jnp.maximum(mx_ref[...], blkmax)


def _k5(o0, o1, b1p, w2p, v2sd):
    f = pl.pallas_call(
        _k5_body,
        grid=(N // ROWBLK,),
        in_specs=[
            pl.BlockSpec((ROWBLK, 128), lambda i: (i, 0)),
            pl.BlockSpec((ROWBLK, 128), lambda i: (i, 0)),
            pl.BlockSpec((1, 256), lambda i: (0, 0)),
            pl.BlockSpec((256, 64), lambda i: (0, 0)),
            pl.BlockSpec((256, 2), lambda i: (0, 0)),
        ],
        out_specs=[
            pl.BlockSpec((ROWBLK, 64), lambda i: (i, 0)),
            pl.BlockSpec((ROWBLK, 16), lambda i: (i, 0)),
            pl.BlockSpec((ROWBLK, 16), lambda i: (i, 0)),
            pl.BlockSpec((8, 16), lambda i: (0, 0)),
        ],
        out_shape=[
            jax.ShapeDtypeStruct((N, 64), jnp.float32),
            jax.ShapeDtypeStruct((N, 16), jnp.float32),
            jax.ShapeDtypeStruct((N, 16), jnp.float32),
            jax.ShapeDtypeStruct((8, 16), jnp.float32),
        ],
    )
    return f(o0, o1, b1p, w2p, v2sd)


def _k7b_body(d0_ref, d1_ref, r_ref):
    r_ref[...] = 1.0 / (d0_ref[...] + d1_ref[...] + 1e-16)


def _k7b(d0, d1):
    f = pl.pallas_call(
        _k7b_body,
        grid=(NP // RPT,),
        in_specs=[
            pl.BlockSpec((RPT, 16), lambda i: (i, 0)),
            pl.BlockSpec((RPT, 16), lambda i: (i, 0)),
        ],
        out_specs=pl.BlockSpec((RPT, 16), lambda i: (i, 0)),
        out_shape=jax.ShapeDtypeStruct((NP, 16), jnp.float32),
    )
    return f(d0, d1)


def _k9_body(p0_ref, p1_ref, b2_ref, o_ref):
    z = p0_ref[...] + p1_ref[...] + b2_ref[...]
    m = jnp.max(z, axis=1, keepdims=True)
    ex = jnp.exp(z - m)
    s = jnp.sum(ex, axis=1, keepdims=True)
    o_ref[...] = z - m - jnp.log(s)


def _k9(p0, p1, b2):
    f = pl.pallas_call(
        _k9_body,
        grid=(N // ROWBLK,),
        in_specs=[
            pl.BlockSpec((ROWBLK, 64), lambda i: (i, 0)),
            pl.BlockSpec((ROWBLK, 64), lambda i: (i, 0)),
            pl.BlockSpec((1, 64), lambda i: (0, 0)),
        ],
        out_specs=pl.BlockSpec((ROWBLK, 64), lambda i: (i, 0)),
        out_shape=jax.ShapeDtypeStruct((N, 64), jnp.float32),
    )
    return f(p0, p1, b2)


def _zero16(buf_ref, rows):
    z = jnp.zeros((16,), jnp.float32)

    def row(i, _):
        buf_ref[i] = z
        return 0

    lax.fori_loop(0, rows, row, 0)


@functools.partial(
    pl.kernel,
    out_type=[
        jax.ShapeDtypeStruct((E, 16), jnp.float32),
        jax.ShapeDtypeStruct((E, 16), jnp.float32),
        jax.ShapeDtypeStruct((NP, 16), jnp.float32),
        jax.ShapeDtypeStruct((NP, 16), jnp.float32),
    ],
    mesh=_MESH,
    compiler_params=_SC_PARAMS,
    scratch_types=[
        pltpu.VMEM((CE,), jnp.int32),
        pltpu.VMEM((CE,), jnp.int32),
        pltpu.VMEM((CE, 16), jnp.float32),
        pltpu.VMEM((CE, 16), jnp.float32),
        pltpu.VMEM((CE, 16), jnp.float32),
        pltpu.VMEM((RPT, 16), jnp.float32),
        pltpu.VMEM((8, 16), jnp.float32),
        pltpu.VMEM_SHARED((NP, 16), jnp.float32),
        pltpu.SemaphoreType.DMA,
    ],
)
def _k2(as0, ad0, as1, ad1, mx1, src_r, dst_r, ex0, ex1, rec0, rec1,
        idxa, idxb, ga, gb, eb, zb, mxb, den_sh, sem):
    c = lax.axis_index("c")
    t = lax.axis_index("s")
    nch = EPT1 // CE

    pltpu.sync_copy(mx1, mxb)
    _zero16(zb, RPT)
    pltpu.sync_copy(zb, den_sh.at[pl.ds(t * RPT, RPT)])
    plsc.subcore_barrier()

    def half(s, as_ref, ad_ref, ex_ref, rec_ref):
        shift = jnp.maximum(mxb[2 * s] + mxb[2 * s + 1], 0.0)

        def chunk(j, _):
            base = t * EPT1 + j * CE
            pltpu.sync_copy(src_r.at[pl.ds(base, CE)], idxa)
            pltpu.async_copy(as_ref.at[idxa], ga, sem).wait()
            pltpu.sync_copy(dst_r.at[pl.ds(base, CE)], idxb)
            pltpu.async_copy(ad_ref.at[idxb], gb, sem).wait()

            def row(i, _):
                e = ga[i] + gb[i]
                e = jnp.where(e >= 0.0, e, 0.2 * e)
                eb[i] = jnp.exp(e - shift)
                return 0

            lax.fori_loop(0, CE, row, 0)
            pltpu.sync_copy(eb, ex_ref.at[pl.ds(base, CE)])
            pltpu.sync_copy(eb, den_sh.at[idxb], add=True)
            return 0

        lax.fori_loop(0, nch, chunk, 0)
        plsc.subcore_barrier()

        r0 = t * RPT
        pltpu.sync_copy(den_sh.at[pl.ds(r0, RPT)], zb)

        def rrow(i, _):
            zb[i] = 1.0 / (zb[i] + 1e-16)
            return 0

        lax.fori_loop(0, RPT, rrow, 0)
        pltpu.sync_copy(zb, rec_ref.at[pl.ds(r0, RPT)])

    @pl.when(c == 0)
    def _():
        half(0, as0, ad0, ex0, rec0)

    @pl.when(c == 1)
    def _():
        half(1, as1, ad1, ex1, rec1)


@functools.partial(
    pl.kernel,
    out_type=[
        jax.ShapeDtypeStruct((NP, 128), jnp.float32),
        jax.ShapeDtypeStruct((NP, 128), jnp.float32),
    ],
    mesh=_MESH,
    compiler_params=_SC_PARAMS,
    scratch_types=[
        pltpu.VMEM((CM3,), jnp.int32),
        pltpu.VMEM((CM3,), jnp.int32),
        pltpu.VMEM((CM3,), jnp.int32),
        pltpu.VMEM((CM3,), jnp.int32),
        pltpu.VMEM((CM3, 16), jnp.float32),
        pltpu.VMEM((CM3, 16), jnp.float32),
        pltpu.VMEM((CM3, 16), jnp.float32),
        pltpu.VMEM((CM3, 16), jnp.float32),
        pltpu.VMEM((CM3, 128), jnp.float32),
        pltpu.VMEM((CM3, 128), jnp.float32),
        pltpu.VMEM((64, 128), jnp.float32),
        pltpu.VMEM_SHARED((NP, 128), jnp.float32),
        pltpu.SemaphoreType.DMA,
        pltpu.SemaphoreType.DMA,
    ],
)
def _k3(h0, h1f, ex0, ex1, rec0, rec1, src_r, dst_r, out0, out1,
        idxs0, idxd0, idxs1, idxd1, exb0, recb0, exb1, recb1,
        hb0, hb1, zb, osh, sem0, sem1):
    c = lax.axis_index("c")
    t = lax.axis_index("s")
    nch = EPT1 // CM3
    z = jnp.zeros((16,), jnp.float32)

    def zrow(i, _):
        for v in range(8):
            zb[i, pl.ds(v * 16, 16)] = z
        return 0

    lax.fori_loop(0, 64, zrow, 0)
    for k in range(RPT // 64):
        pltpu.sync_copy(zb, osh.at[pl.ds(t * RPT + k * 64, 64)])
    plsc.subcore_barrier()

    def half(h_ref, ex_ref, rec_ref, out_ref):
        bufs = ((idxs0, idxd0, hb0, recb0, exb0, sem0),
                (idxs1, idxd1, hb1, recb1, exb1, sem1))

        def prep(j, b):
            idxs, idxd, hb, recb, exb, sem = bufs[b]
            base = t * EPT1 + j * CM3
            pltpu.sync_copy(src_r.at[pl.ds(base, CM3)], idxs)
            pltpu.async_copy(h_ref.at[idxs], hb, sem)
            pltpu.sync_copy(dst_r.at[pl.ds(base, CM3)], idxd)
            pltpu.async_copy(rec_ref.at[idxd], recb, sem)
            pltpu.async_copy(ex_ref.at[pl.ds(base, CM3)], exb, sem)

        def consume(b):
            idxs, idxd, hb, recb, exb, sem = bufs[b]
            pltpu.make_async_copy(h_ref.at[idxs], hb, sem).wait()
            pltpu.make_async_copy(rec_ref.at[idxd], recb, sem).wait()
            pltpu.make_async_copy(ex_ref.at[pl.ds(0, CM3)], exb, sem).wait()

            def row(i, _):
                alpha = exb[i] * recb[i]
                for v in range(8):
                    sl = pl.ds(v * 16, 16)
                    hb[i, sl] = hb[i, sl] * alpha
                return 0

            lax.fori_loop(0, CM3, row, 0)
            pltpu.sync_copy(hb, osh.at[idxd], add=True)

        prep(0, 0)
        prep(1, 1)

        def pair(j2, _):
            consume(0)

            @pl.when(j2 < nch // 2 - 1)
            def _():
                prep(2 * j2 + 2, 0)

            consume(1)

            @pl.when(j2 < nch // 2 - 1)
            def _():
                prep(2 * j2 + 3, 1)

            return 0

        lax.fori_loop(0, nch // 2, pair, 0)
        plsc.subcore_barrier()
        r0 = t * RPT
        pltpu.sync_copy(osh.at[pl.ds(r0, RPT)], out_ref.at[pl.ds(r0, RPT)])

    @pl.when(c == 0)
    def _():
        half(h0, ex0, rec0, out0)

    @pl.when(c == 1)
    def _():
        half(h1f, ex1, rec1, out1)


@functools.partial(
    pl.kernel,
    out_type=[
        jax.ShapeDtypeStruct((E, 16), jnp.float32),
        jax.ShapeDtypeStruct((NP, 16), jnp.float32),
        jax.ShapeDtypeStruct((NP, 16), jnp.float32),
    ],
    mesh=_MESH,
    compiler_params=_SC_PARAMS,
    scratch_types=[
        pltpu.VMEM((CE,), jnp.int32),
        pltpu.VMEM((CE,), jnp.int32),
        pltpu.VMEM((CE, 16), jnp.float32),
        pltpu.VMEM((CE, 16), jnp.float32),
        pltpu.VMEM((CE, 16), jnp.float32),
        pltpu.VMEM((RPT, 16), jnp.float32),
        pltpu.VMEM((8, 16), jnp.float32),
        pltpu.VMEM_SHARED((NP, 16), jnp.float32),
        pltpu.SemaphoreType.DMA,
    ],
)
def _k7(a2s, a2d, mx2, src_r, dst_r, ex2s, d20, d21,
        idxa, idxb, ga, gb, eb, zb, mxb, den_sh, sem):
    c = lax.axis_index("c")
    t = lax.axis_index("s")
    nch = EPT2 // CE

    pltpu.sync_copy(mx2, mxb)
    shift = jnp.maximum(mxb[0] + mxb[1], 0.0)

    _zero16(zb, RPT)
    pltpu.sync_copy(zb, den_sh.at[pl.ds(t * RPT, RPT)])
    plsc.subcore_barrier()

    def chunk(j, _):
        base = c * (E // 2) + t * EPT2 + j * CE
        pltpu.sync_copy(src_r.at[pl.ds(base, CE)], idxa)
        pltpu.async_copy(a2s.at[idxa], ga, sem).wait()
        pltpu.sync_copy(dst_r.at[pl.ds(base, CE)], idxb)
        pltpu.async_copy(a2d.at[idxb], gb, sem).wait()

        def row(i, _):
            e = ga[i] + gb[i]
            e = jnp.where(e >= 0.0, e, 0.2 * e)
            eb[i] = jnp.exp(e - shift)
            return 0

        lax.fori_loop(0, CE, row, 0)
        pltpu.sync_copy(eb, ex2s.at[pl.ds(base, CE)])
        pltpu.sync_copy(eb, den_sh.at[idxb], add=True)
        return 0

    lax.fori_loop(0, nch, chunk, 0)
    plsc.subcore_barrier()

    r0 = t * RPT

    @pl.when(c == 0)
    def _():
        pltpu.sync_copy(den_sh.at[pl.ds(r0, RPT)], d20.at[pl.ds(r0, RPT)])

    @pl.when(c == 1)
    def _():
        pltpu.sync_copy(den_sh.at[pl.ds(r0, RPT)], d21.at[pl.ds(r0, RPT)])


@functools.partial(
    pl.kernel,
    out_type=[
        jax.ShapeDtypeStruct((NP, 64), jnp.float32),
        jax.ShapeDtypeStruct((NP, 64), jnp.float32),
    ],
    mesh=_MESH,
    compiler_params=_SC_PARAMS,
    scratch_types=[
        pltpu.VMEM((CM,), jnp.int32),
        pltpu.VMEM((CM,), jnp.int32),
        pltpu.VMEM((CM,), jnp.int32),
        pltpu.VMEM((CM,), jnp.int32),
        pltpu.VMEM((CM, 16), jnp.float32),
        pltpu.VMEM((CM, 16), jnp.float32),
        pltpu.VMEM((CM, 16), jnp.float32),
        pltpu.VMEM((CM, 16), jnp.float32),
        pltpu.VMEM((CM, 64), jnp.float32),
        pltpu.VMEM((CM, 64), jnp.float32),
        pltpu.VMEM((128, 64), jnp.float32),
        pltpu.VMEM_SHARED((NP, 64), jnp.float32),
        pltpu.SemaphoreType.DMA,
        pltpu.SemaphoreType.DMA,
    ],
)
def _k8(h2, ex2s, rec2, src_r, dst_r, o2p0, o2p1,
        idxs0, idxd0, idxs1, idxd1, exb0, recb0, exb1, recb1,
        hb0, hb1, zb, osh, sem0, sem1):
    c = lax.axis_index("c")
    t = lax.axis_index("s")
    nch = EPT2 // CM
    z = jnp.zeros((16,), jnp.float32)

    def zrow(i, _):
        for v in range(4):
            zb[i, pl.ds(v * 16, 16)] = z
        return 0

    lax.fori_loop(0, 128, zrow, 0)
    for k in range(RPT // 128):
        pltpu.sync_copy(zb, osh.at[pl.ds(t * RPT + k * 128, 128)])
    plsc.subcore_barrier()

    bufs = ((idxs0, idxd0, hb0, recb0, exb0, sem0),
            (idxs1, idxd1, hb1, recb1, exb1, sem1))

    def prep(j, b):
        idxs, idxd, hb, recb, exb, sem = bufs[b]
        base = c * (E // 2) + t * EPT2 + j * CM
        pltpu.sync_copy(src_r.at[pl.ds(base, CM)], idxs)
        pltpu.async_copy(h2.at[idxs], hb, sem)
        pltpu.sync_copy(dst_r.at[pl.ds(base, CM)], idxd)
        pltpu.async_copy(rec2.at[idxd], recb, sem)
        pltpu.async_copy(ex2s.at[pl.ds(base, CM)], exb, sem)

    def consume(b):
        idxs, idxd, hb, recb, exb, sem = bufs[b]
        pltpu.make_async_copy(h2.at[idxs], hb, sem).wait()
        pltpu.make_async_copy(rec2.at[idxd], recb, sem).wait()
        pltpu.make_async_copy(ex2s.at[pl.ds(0, CM)], exb, sem).wait()

        def row(i, _):
            alpha = exb[i] * recb[i]
            for v in range(4):
                sl = pl.ds(v * 16, 16)
                hb[i, sl] = hb[i, sl] * alpha
            return 0

        lax.fori_loop(0, CM, row, 0)
        pltpu.sync_copy(hb, osh.at[idxd], add=True)

    prep(0, 0)
    prep(1, 1)

    def pair(j2, _):
        consume(0)

        @pl.when(j2 < nch // 2 - 1)
        def _():
            prep(2 * j2 + 2, 0)

        consume(1)

        @pl.when(j2 < nch // 2 - 1)
        def _():
            prep(2 * j2 + 3, 1)

        return 0

    lax.fori_loop(0, nch // 2, pair, 0)
    plsc.subcore_barrier()

    r0 = t * RPT

    @pl.when(c == 0)
    def _():
        pltpu.sync_copy(osh.at[pl.ds(r0, RPT)], o2p0.at[pl.ds(r0, RPT)])

    @pl.when(c == 1)
    def _():
        pltpu.sync_copy(osh.at[pl.ds(r0, RPT)], o2p1.at[pl.ds(r0, RPT)])


def kernel(x, edge_index, W1, att_src1, att_dst1, b1, W2, att_src2, att_dst2, b2):
    src = edge_index[0]
    dst = edge_index[1]

    perm = jnp.asarray(_PERM)
    w1p = W1[:, perm]
    v_src = jnp.sum(W1.reshape(128, H1, C1) * att_src1[None], axis=-1)
    v_dst = jnp.sum(W1.reshape(128, H1, C1) * att_dst1[None], axis=-1)
    rep = jnp.asarray(np.tile(np.arange(4), 4))
    wbig = jnp.concatenate(
        [w1p,
         v_src[:, 0 + rep], v_dst[:, 0 + rep],
         v_src[:, 4 + rep], v_dst[:, 4 + rep]], axis=1)
    b1p = b1[perm][None, :]
    w2p = W2[perm, :]
    v2sd = jnp.stack([w2p @ att_src2[0], w2p @ att_dst2[0]], axis=1)

    h0, h1f, as0, ad0, as1, ad1, mx1 = _k1(x, wbig)
    ex0, ex1, rec0, rec1 = _k2(as0, ad0, as1, ad1, mx1, src, dst)
    out0, out1 = _k3(h0, h1f, ex0, ex1, rec0, rec1, src, dst)

    h2, a2s_t, a2d_t, mx2 = _k5(out0, out1, b1p, w2p, v2sd)
    ex2s, d20, d21 = _k7(a2s_t, a2d_t, mx2, src, dst)
    rec2 = _k7b(d20, d21)
    o2p0, o2p1 = _k8(h2, ex2s, rec2, src, dst)
    return _k9(o2p0, o2p1, b2[None, :])


# speedup vs baseline: 51.2794x; 1.2119x over previous
import dataclasses
import functools

import jax
import jax.numpy as jnp
import numpy as np
from jax import lax
from jax.experimental import pallas as pl
from jax.experimental.pallas import tpu as pltpu
from jax.experimental.pallas import tpu_sc as plsc

N = 10000
E = 320000
NP = 10240
H1 = 8
C1 = 32
ROWBLK = 400

NSUB = 16
RPT = NP // NSUB

EPT1 = E // NSUB
EPT2 = E // (2 * NSUB)
CE = 1000
CM = 200
CM3 = 80

_MESH = plsc.VectorSubcoreMesh(core_axis_name="c", subcore_axis_name="s")

_SC_PARAMS = pltpu.CompilerParams()
if "needs_layout_passes" in pltpu.CompilerParams.__dataclass_fields__:
    _SC_PARAMS = dataclasses.replace(
        _SC_PARAMS, needs_layout_passes=False, use_tc_tiling_on_sc=False)


def _perm_half(s):
    p = np.arange(128)
    return (4 * s + (p % 4)) * C1 + (p // 4)


_PERM = np.concatenate([_perm_half(0), _perm_half(1)])


def _k1_body(x_ref, w_ref, h0_ref, h1_ref, as0_ref, ad0_ref, as1_ref, ad1_ref,
             mx_ref):
    i = pl.program_id(0)
    y = jnp.dot(x_ref[...], w_ref[...], preferred_element_type=jnp.float32)
    h0_ref[...] = y[:, 0:128]
    h1_ref[...] = y[:, 128:256]
    as0_ref[...] = y[:, 256:272]
    ad0_ref[...] = y[:, 272:288]
    as1_ref[...] = y[:, 288:304]
    ad1_ref[...] = y[:, 304:320]
    blkmax = jnp.concatenate(
        [jnp.max(y[:, 256 + 16 * k:272 + 16 * k], axis=0)[None] for k in range(4)]
        + [jnp.full((4, 16), -3.0e38, jnp.float32)], axis=0)

    @pl.when(i == 0)
    def _():
        mx_ref[...] = jnp.full((8, 16), -3.0e38, jnp.float32)

    mx_ref[...] = jnp.maximum(mx_ref[...], blkmax)


def _k1(x, wbig):
    f = pl.pallas_call(
        _k1_body,
        grid=(N // ROWBLK,),
        in_specs=[
            pl.BlockSpec((ROWBLK, 128), lambda i: (i, 0)),
            pl.BlockSpec((128, 320), lambda i: (0, 0)),
        ],
        out_specs=[
            pl.BlockSpec((ROWBLK, 128), lambda i: (i, 0)),
            pl.BlockSpec((ROWBLK, 128), lambda i: (i, 0)),
            pl.BlockSpec((ROWBLK, 16), lambda i: (i, 0)),
            pl.BlockSpec((ROWBLK, 16), lambda i: (i, 0)),
            pl.BlockSpec((ROWBLK, 16), lambda i: (i, 0)),
            pl.BlockSpec((ROWBLK, 16), lambda i: (i, 0)),
            pl.BlockSpec((8, 16), lambda i: (0, 0)),
        ],
        out_shape=[
            jax.ShapeDtypeStruct((N, 128), jnp.float32),
            jax.ShapeDtypeStruct((N, 128), jnp.float32),
            jax.ShapeDtypeStruct((N, 16), jnp.float32),
            jax.ShapeDtypeStruct((N, 16), jnp.float32),
            jax.ShapeDtypeStruct((N, 16), jnp.float32),
            jax.ShapeDtypeStruct((N, 16), jnp.float32),
            jax.ShapeDtypeStruct((8, 16), jnp.float32),
        ],
    )
    return f(x, wbig)


def _k5_body(o0_ref, o1_ref, b1_ref, w2_ref, v2_ref, h2_ref, a2s_ref, a2d_ref,
             mx_ref):
    i = pl.program_id(0)
    h1 = jnp.concatenate([o0_ref[...], o1_ref[...]], axis=1) + b1_ref[...]
    h1 = jnp.where(h1 > 0, h1, jnp.exp(jnp.minimum(h1, 0.0)) - 1.0)
    h2_ref[...] = jnp.dot(h1, w2_ref[...], preferred_element_type=jnp.float32)
    sd = jnp.dot(h1, v2_ref[...], preferred_element_type=jnp.float32)
    a2s = jnp.broadcast_to(sd[:, 0:1], (sd.shape[0], 16))
    a2d = jnp.broadcast_to(sd[:, 1:2], (sd.shape[0], 16))
    a2s_ref[...] = a2s
    a2d_ref[...] = a2d
    blkmax = jnp.concatenate(
        [jnp.max(a2s, axis=0)[None], jnp.max(a2d, axis=0)[None],
         jnp.full((6, 16), -3.0e38, jnp.float32)], axis=0)

    @pl.when(i == 0)
    def _():
        mx_ref[...] = jnp.full((8, 16), -3.0e38, jnp.float32)

    mx_ref[...] = jnp.maximum(mx_ref[...], blkmax)


def _k5(o0, o1, b1p, w2p, v2sd):
    f = pl.pallas_call(
        _k5_body,
        grid=(N // ROWBLK,),
        in_specs=[
            pl.BlockSpec((ROWBLK, 128), lambda i: (i, 0)),
            pl.BlockSpec((ROWBLK, 128), lambda i: (i, 0)),
            pl.BlockSpec((1, 256), lambda i: (0, 0)),
            pl.BlockSpec((256, 64), lambda i: (0, 0)),
            pl.BlockSpec((256, 2), lambda i: (0, 0)),
        ],
        out_specs=[
            pl.BlockSpec((ROWBLK, 64), lambda i: (i, 0)),
            pl.BlockSpec((ROWBLK, 16), lambda i: (i, 0)),
            pl.BlockSpec((ROWBLK, 16), lambda i: (i, 0)),
            pl.BlockSpec((8, 16), lambda i: (0, 0)),
        ],
        out_shape=[
            jax.ShapeDtypeStruct((N, 64), jnp.float32),
            jax.ShapeDtypeStruct((N, 16), jnp.float32),
            jax.ShapeDtypeStruct((N, 16), jnp.float32),
            jax.ShapeDtypeStruct((8, 16), jnp.float32),
        ],
    )
    return f(o0, o1, b1p, w2p, v2sd)


def _k7b_body(d0_ref, d1_ref, r_ref):
    r_ref[...] = 1.0 / (d0_ref[...] + d1_ref[...] + 1e-16)


def _k7b(d0, d1):
    f = pl.pallas_call(
        _k7b_body,
        grid=(NP // RPT,),
        in_specs=[
            pl.BlockSpec((RPT, 16), lambda i: (i, 0)),
            pl.BlockSpec((RPT, 16), lambda i: (i, 0)),
        ],
        out_specs=pl.BlockSpec((RPT, 16), lambda i: (i, 0)),
        out_shape=jax.ShapeDtypeStruct((NP, 16), jnp.float32),
    )
    return f(d0, d1)


def _k9_body(p0_ref, p1_ref, b2_ref, o_ref):
    z = p0_ref[...] + p1_ref[...] + b2_ref[...]
    m = jnp.max(z, axis=1, keepdims=True)
    ex = jnp.exp(z - m)
    s = jnp.sum(ex, axis=1, keepdims=True)
    o_ref[...] = z - m - jnp.log(s)


def _k9(p0, p1, b2):
    f = pl.pallas_call(
        _k9_body,
        grid=(N // ROWBLK,),
        in_specs=[
            pl.BlockSpec((ROWBLK, 64), lambda i: (i, 0)),
            pl.BlockSpec((ROWBLK, 64), lambda i: (i, 0)),
            pl.BlockSpec((1, 64), lambda i: (0, 0)),
        ],
        out_specs=pl.BlockSpec((ROWBLK, 64), lambda i: (i, 0)),
        out_shape=jax.ShapeDtypeStruct((N, 64), jnp.float32),
    )
    return f(p0, p1, b2)


def _zero16(buf_ref, rows):
    z = jnp.zeros((16,), jnp.float32)

    def row(i, _):
        buf_ref[i] = z
        return 0

    lax.fori_loop(0, rows, row, 0)


@functools.partial(
    pl.kernel,
    out_type=[
        jax.ShapeDtypeStruct((E, 16), jnp.float32),
        jax.ShapeDtypeStruct((E, 16), jnp.float32),
        jax.ShapeDtypeStruct((NP, 16), jnp.float32),
        jax.ShapeDtypeStruct((NP, 16), jnp.float32),
    ],
    mesh=_MESH,
    compiler_params=_SC_PARAMS,
    scratch_types=[
        pltpu.VMEM((CE,), jnp.int32),
        pltpu.VMEM((CE,), jnp.int32),
        pltpu.VMEM((CE, 16), jnp.float32),
        pltpu.VMEM((CE, 16), jnp.float32),
        pltpu.VMEM((CE, 16), jnp.float32),
        pltpu.VMEM((RPT, 16), jnp.float32),
        pltpu.VMEM((8, 16), jnp.float32),
        pltpu.VMEM_SHARED((NP, 16), jnp.float32),
        pltpu.SemaphoreType.DMA,
    ],
)
def _k2(as0, ad0, as1, ad1, mx1, src_r, dst_r, ex0, ex1, rec0, rec1,
        idxa, idxb, ga, gb, eb, zb, mxb, den_sh, sem):
    c = lax.axis_index("c")
    t = lax.axis_index("s")
    nch = EPT1 // CE

    pltpu.sync_copy(mx1, mxb)
    _zero16(zb, RPT)
    pltpu.sync_copy(zb, den_sh.at[pl.ds(t * RPT, RPT)])
    plsc.subcore_barrier()

    def half(s, as_ref, ad_ref, ex_ref, rec_ref):
        shift = jnp.maximum(mxb[2 * s] + mxb[2 * s + 1], 0.0)

        def chunk(j, _):
            base = t * EPT1 + j * CE
            pltpu.sync_copy(src_r.at[pl.ds(base, CE)], idxa)
            pltpu.async_copy(as_ref.at[idxa], ga, sem).wait()
            pltpu.sync_copy(dst_r.at[pl.ds(base, CE)], idxb)
            pltpu.async_copy(ad_ref.at[idxb], gb, sem).wait()

            @plsc.parallel_loop(0, CE, unroll=4)
            def _(i):
                e = ga[i] + gb[i]
                e = jnp.where(e >= 0.0, e, 0.2 * e)
                eb[i] = jnp.exp(e - shift)
            pltpu.sync_copy(eb, ex_ref.at[pl.ds(base, CE)])
            pltpu.sync_copy(eb, den_sh.at[idxb], add=True)
            return 0

        lax.fori_loop(0, nch, chunk, 0)
        plsc.subcore_barrier()

        r0 = t * RPT
        pltpu.sync_copy(den_sh.at[pl.ds(r0, RPT)], zb)

        def rrow(i, _):
            zb[i] = 1.0 / (zb[i] + 1e-16)
            return 0

        lax.fori_loop(0, RPT, rrow, 0)
        pltpu.sync_copy(zb, rec_ref.at[pl.ds(r0, RPT)])

    @pl.when(c == 0)
    def _():
        half(0, as0, ad0, ex0, rec0)

    @pl.when(c == 1)
    def _():
        half(1, as1, ad1, ex1, rec1)


@functools.partial(
    pl.kernel,
    out_type=[
        jax.ShapeDtypeStruct((NP, 128), jnp.float32),
        jax.ShapeDtypeStruct((NP, 128), jnp.float32),
    ],
    mesh=_MESH,
    compiler_params=_SC_PARAMS,
    scratch_types=[
        pltpu.VMEM((CM3,), jnp.int32),
        pltpu.VMEM((CM3,), jnp.int32),
        pltpu.VMEM((CM3,), jnp.int32),
        pltpu.VMEM((CM3,), jnp.int32),
        pltpu.VMEM((CM3, 16), jnp.float32),
        pltpu.VMEM((CM3, 16), jnp.float32),
        pltpu.VMEM((CM3, 16), jnp.float32),
        pltpu.VMEM((CM3, 16), jnp.float32),
        pltpu.VMEM((CM3, 128), jnp.float32),
        pltpu.VMEM((CM3, 128), jnp.float32),
        pltpu.VMEM((64, 128), jnp.float32),
        pltpu.VMEM_SHARED((NP, 128), jnp.float32),
        pltpu.SemaphoreType.DMA,
        pltpu.SemaphoreType.DMA,
    ],
)
def _k3(h0, h1f, ex0, ex1, rec0, rec1, src_r, dst_r, out0, out1,
        idxs0, idxd0, idxs1, idxd1, exb0, recb0, exb1, recb1,
        hb0, hb1, zb, osh, sem0, sem1):
    c = lax.axis_index("c")
    t = lax.axis_index("s")
    nch = EPT1 // CM3
    z = jnp.zeros((16,), jnp.float32)

    def zrow(i, _):
        for v in range(8):
            zb[i, pl.ds(v * 16, 16)] = z
        return 0

    lax.fori_loop(0, 64, zrow, 0)
    for k in range(RPT // 64):
        pltpu.sync_copy(zb, osh.at[pl.ds(t * RPT + k * 64, 64)])
    plsc.subcore_barrier()

    def half(h_ref, ex_ref, rec_ref, out_ref):
        bufs = ((idxs0, idxd0, hb0, recb0, exb0, sem0),
                (idxs1, idxd1, hb1, recb1, exb1, sem1))

        def prep(j, b):
            idxs, idxd, hb, recb, exb, sem = bufs[b]
            base = t * EPT1 + j * CM3
            pltpu.sync_copy(src_r.at[pl.ds(base, CM3)], idxs)
            pltpu.async_copy(h_ref.at[idxs], hb, sem)
            pltpu.sync_copy(dst_r.at[pl.ds(base, CM3)], idxd)
            pltpu.async_copy(rec_ref.at[idxd], recb, sem)
            pltpu.async_copy(ex_ref.at[pl.ds(base, CM3)], exb, sem)

        def consume(b):
            idxs, idxd, hb, recb, exb, sem = bufs[b]
            pltpu.make_async_copy(h_ref.at[idxs], hb, sem).wait()
            pltpu.make_async_copy(rec_ref.at[idxd], recb, sem).wait()
            pltpu.make_async_copy(ex_ref.at[pl.ds(0, CM3)], exb, sem).wait()

            @plsc.parallel_loop(0, CM3, unroll=4)
            def _(i):
                alpha = exb[i] * recb[i]
                for v in range(8):
                    sl = pl.ds(v * 16, 16)
                    hb[i, sl] = hb[i, sl] * alpha
            pltpu.sync_copy(hb, osh.at[idxd], add=True)

        prep(0, 0)
        prep(1, 1)

        def pair(j2, _):
            consume(0)

            @pl.when(j2 < nch // 2 - 1)
            def _():
                prep(2 * j2 + 2, 0)

            consume(1)

            @pl.when(j2 < nch // 2 - 1)
            def _():
                prep(2 * j2 + 3, 1)

            return 0

        lax.fori_loop(0, nch // 2, pair, 0)
        plsc.subcore_barrier()
        r0 = t * RPT
        pltpu.sync_copy(osh.at[pl.ds(r0, RPT)], out_ref.at[pl.ds(r0, RPT)])

    @pl.when(c == 0)
    def _():
        half(h0, ex0, rec0, out0)

    @pl.when(c == 1)
    def _():
        half(h1f, ex1, rec1, out1)


@functools.partial(
    pl.kernel,
    out_type=[
        jax.ShapeDtypeStruct((E, 16), jnp.float32),
        jax.ShapeDtypeStruct((NP, 16), jnp.float32),
        jax.ShapeDtypeStruct((NP, 16), jnp.float32),
    ],
    mesh=_MESH,
    compiler_params=_SC_PARAMS,
    scratch_types=[
        pltpu.VMEM((CE,), jnp.int32),
        pltpu.VMEM((CE,), jnp.int32),
        pltpu.VMEM((CE, 16), jnp.float32),
        pltpu.VMEM((CE, 16), jnp.float32),
        pltpu.VMEM((CE, 16), jnp.float32),
        pltpu.VMEM((RPT, 16), jnp.float32),
        pltpu.VMEM((8, 16), jnp.float32),
        pltpu.VMEM_SHARED((NP, 16), jnp.float32),
        pltpu.SemaphoreType.DMA,
    ],
)
def _k7(a2s, a2d, mx2, src_r, dst_r, ex2s, d20, d21,
        idxa, idxb, ga, gb, eb, zb, mxb, den_sh, sem):
    c = lax.axis_index("c")
    t = lax.axis_index("s")
    nch = EPT2 // CE

    pltpu.sync_copy(mx2, mxb)
    shift = jnp.maximum(mxb[0] + mxb[1], 0.0)

    _zero16(zb, RPT)
    pltpu.sync_copy(zb, den_sh.at[pl.ds(t * RPT, RPT)])
    plsc.subcore_barrier()

    def chunk(j, _):
        base = c * (E // 2) + t * EPT2 + j * CE
        pltpu.sync_copy(src_r.at[pl.ds(base, CE)], idxa)
        pltpu.async_copy(a2s.at[idxa], ga, sem).wait()
        pltpu.sync_copy(dst_r.at[pl.ds(base, CE)], idxb)
        pltpu.async_copy(a2d.at[idxb], gb, sem).wait()

        @plsc.parallel_loop(0, CE, unroll=4)
        def _(i):
            e = ga[i] + gb[i]
            e = jnp.where(e >= 0.0, e, 0.2 * e)
            eb[i] = jnp.exp(e - shift)
        pltpu.sync_copy(eb, ex2s.at[pl.ds(base, CE)])
        pltpu.sync_copy(eb, den_sh.at[idxb], add=True)
        return 0

    lax.fori_loop(0, nch, chunk, 0)
    plsc.subcore_barrier()

    r0 = t * RPT

    @pl.when(c == 0)
    def _():
        pltpu.sync_copy(den_sh.at[pl.ds(r0, RPT)], d20.at[pl.ds(r0, RPT)])

    @pl.when(c == 1)
    def _():
        pltpu.sync_copy(den_sh.at[pl.ds(r0, RPT)], d21.at[pl.ds(r0, RPT)])


@functools.partial(
    pl.kernel,
    out_type=[
        jax.ShapeDtypeStruct((NP, 64), jnp.float32),
        jax.ShapeDtypeStruct((NP, 64), jnp.float32),
    ],
    mesh=_MESH,
    compiler_params=_SC_PARAMS,
    scratch_types=[
        pltpu.VMEM((CM,), jnp.int32),
        pltpu.VMEM((CM,), jnp.int32),
        pltpu.VMEM((CM,), jnp.int32),
        pltpu.VMEM((CM,), jnp.int32),
        pltpu.VMEM((CM, 16), jnp.float32),
        pltpu.VMEM((CM, 16), jnp.float32),
        pltpu.VMEM((CM, 16), jnp.float32),
        pltpu.VMEM((CM, 16), jnp.float32),
        pltpu.VMEM((CM, 64), jnp.float32),
        pltpu.VMEM((CM, 64), jnp.float32),
        pltpu.VMEM((128, 64), jnp.float32),
        pltpu.VMEM_SHARED((NP, 64), jnp.float32),
        pltpu.SemaphoreType.DMA,
        pltpu.SemaphoreType.DMA,
    ],
)
def _k8(h2, ex2s, rec2, src_r, dst_r, o2p0, o2p1,
        idxs0, idxd0, idxs1, idxd1, exb0, recb0, exb1, recb1,
        hb0, hb1, zb, osh, sem0, sem1):
    c = lax.axis_index("c")
    t = lax.axis_index("s")
    nch = EPT2 // CM
    z = jnp.zeros((16,), jnp.float32)

    def zrow(i, _):
        for v in range(4):
            zb[i, pl.ds(v * 16, 16)] = z
        return 0

    lax.fori_loop(0, 128, zrow, 0)
    for k in range(RPT // 128):
        pltpu.sync_copy(zb, osh.at[pl.ds(t * RPT + k * 128, 128)])
    plsc.subcore_barrier()

    bufs = ((idxs0, idxd0, hb0, recb0, exb0, sem0),
            (idxs1, idxd1, hb1, recb1, exb1, sem1))

    def prep(j, b):
        idxs, idxd, hb, recb, exb, sem = bufs[b]
        base = c * (E // 2) + t * EPT2 + j * CM
        pltpu.sync_copy(src_r.at[pl.ds(base, CM)], idxs)
        pltpu.async_copy(h2.at[idxs], hb, sem)
        pltpu.sync_copy(dst_r.at[pl.ds(base, CM)], idxd)
        pltpu.async_copy(rec2.at[idxd], recb, sem)
        pltpu.async_copy(ex2s.at[pl.ds(base, CM)], exb, sem)

    def consume(b):
        idxs, idxd, hb, recb, exb, sem = bufs[b]
        pltpu.make_async_copy(h2.at[idxs], hb, sem).wait()
        pltpu.make_async_copy(rec2.at[idxd], recb, sem).wait()
        pltpu.make_async_copy(ex2s.at[pl.ds(0, CM)], exb, sem).wait()

        @plsc.parallel_loop(0, CM, unroll=4)
        def _(i):
            alpha = exb[i] * recb[i]
            for v in range(4):
                sl = pl.ds(v * 16, 16)
                hb[i, sl] = hb[i, sl] * alpha
        pltpu.sync_copy(hb, osh.at[idxd], add=True)

    prep(0, 0)
    prep(1, 1)

    def pair(j2, _):
        consume(0)

        @pl.when(j2 < nch // 2 - 1)
        def _():
            prep(2 * j2 + 2, 0)

        consume(1)

        @pl.when(j2 < nch // 2 - 1)
        def _():
            prep(2 * j2 + 3, 1)

        return 0

    lax.fori_loop(0, nch // 2, pair, 0)
    plsc.subcore_barrier()

    r0 = t * RPT

    @pl.when(c == 0)
    def _():
        pltpu.sync_copy(osh.at[pl.ds(r0, RPT)], o2p0.at[pl.ds(r0, RPT)])

    @pl.when(c == 1)
    def _():
        pltpu.sync_copy(osh.at[pl.ds(r0, RPT)], o2p1.at[pl.ds(r0, RPT)])


def kernel(x, edge_index, W1, att_src1, att_dst1, b1, W2, att_src2, att_dst2, b2):
    src = edge_index[0]
    dst = edge_index[1]

    perm = jnp.asarray(_PERM)
    w1p = W1[:, perm]
    v_src = jnp.sum(W1.reshape(128, H1, C1) * att_src1[None], axis=-1)
    v_dst = jnp.sum(W1.reshape(128, H1, C1) * att_dst1[None], axis=-1)
    rep = jnp.asarray(np.tile(np.arange(4), 4))
    wbig = jnp.concatenate(
        [w1p,
         v_src[:, 0 + rep], v_dst[:, 0 + rep],
         v_src[:, 4 + rep], v_dst[:, 4 + rep]], axis=1)
    b1p = b1[perm][None, :]
    w2p = W2[perm, :]
    v2sd = jnp.stack([w2p @ att_src2[0], w2p @ att_dst2[0]], axis=1)

    h0, h1f, as0, ad0, as1, ad1, mx1 = _k1(x, wbig)
    ex0, ex1, rec0, rec1 = _k2(as0, ad0, as1, ad1, mx1, src, dst)
    out0, out1 = _k3(h0, h1f, ex0, ex1, rec0, rec1, src, dst)

    h2, a2s_t, a2d_t, mx2 = _k5(out0, out1, b1p, w2p, v2sd)
    ex2s, d20, d21 = _k7(a2s_t, a2d_t, mx2, src, dst)
    rec2 = _k7b(d20, d21)
    o2p0, o2p1 = _k8(h2, ex2s, rec2, src, dst)
    return _k9(o2p0, o2p1, b2[None, :])


# speedup vs baseline: 57.2746x; 1.1169x over previous
import dataclasses
import functools

import jax
import jax.numpy as jnp
import numpy as np
from jax import lax
from jax.experimental import pallas as pl
from jax.experimental.pallas import tpu as pltpu
from jax.experimental.pallas import tpu_sc as plsc

N = 10000
E = 320000
NP = 10240
H1 = 8
C1 = 32
ROWBLK = 400

NSUB = 16
RPT = NP // NSUB

EPT1 = E // NSUB
EPT2 = E // (2 * NSUB)
CE = 1000
CM = 200
CM3 = 80

_MESH = plsc.VectorSubcoreMesh(core_axis_name="c", subcore_axis_name="s")

_SC_PARAMS = pltpu.CompilerParams()
if "needs_layout_passes" in pltpu.CompilerParams.__dataclass_fields__:
    _SC_PARAMS = dataclasses.replace(
        _SC_PARAMS, needs_layout_passes=False, use_tc_tiling_on_sc=False)


def _perm_half(s):
    p = np.arange(128)
    return (4 * s + (p % 4)) * C1 + (p // 4)


_PERM = np.concatenate([_perm_half(0), _perm_half(1)])


def _k1_body(x_ref, w_ref, h0_ref, h1_ref, as0_ref, ad0_ref, as1_ref, ad1_ref,
             mx_ref):
    i = pl.program_id(0)
    y = jnp.dot(x_ref[...], w_ref[...], preferred_element_type=jnp.float32)
    h0_ref[...] = y[:, 0:128]
    h1_ref[...] = y[:, 128:256]
    as0_ref[...] = y[:, 256:272]
    ad0_ref[...] = y[:, 272:288]
    as1_ref[...] = y[:, 288:304]
    ad1_ref[...] = y[:, 304:320]
    blkmax = jnp.concatenate(
        [jnp.max(y[:, 256 + 16 * k:272 + 16 * k], axis=0)[None] for k in range(4)]
        + [jnp.full((4, 16), -3.0e38, jnp.float32)], axis=0)

    @pl.when(i == 0)
    def _():
        mx_ref[...] = jnp.full((8, 16), -3.0e38, jnp.float32)

    mx_ref[...] = jnp.maximum(mx_ref[...], blkmax)


def _k1(x, wbig):
    f = pl.pallas_call(
        _k1_body,
        grid=(N // ROWBLK,),
        in_specs=[
            pl.BlockSpec((ROWBLK, 128), lambda i: (i, 0)),
            pl.BlockSpec((128, 320), lambda i: (0, 0)),
        ],
        out_specs=[
            pl.BlockSpec((ROWBLK, 128), lambda i: (i, 0)),
            pl.BlockSpec((ROWBLK, 128), lambda i: (i, 0)),
            pl.BlockSpec((ROWBLK, 16), lambda i: (i, 0)),
            pl.BlockSpec((ROWBLK, 16), lambda i: (i, 0)),
            pl.BlockSpec((ROWBLK, 16), lambda i: (i, 0)),
            pl.BlockSpec((ROWBLK, 16), lambda i: (i, 0)),
            pl.BlockSpec((8, 16), lambda i: (0, 0)),
        ],
        out_shape=[
            jax.ShapeDtypeStruct((N, 128), jnp.float32),
            jax.ShapeDtypeStruct((N, 128), jnp.float32),
            jax.ShapeDtypeStruct((N, 16), jnp.float32),
            jax.ShapeDtypeStruct((N, 16), jnp.float32),
            jax.ShapeDtypeStruct((N, 16), jnp.float32),
            jax.ShapeDtypeStruct((N, 16), jnp.float32),
            jax.ShapeDtypeStruct((8, 16), jnp.float32),
        ],
    )
    return f(x, wbig)


def _k5_body(o0_ref, o1_ref, b1_ref, w2_ref, v2_ref, h2_ref, a2s_ref, a2d_ref,
             mx_ref):
    i = pl.program_id(0)
    h1 = jnp.concatenate([o0_ref[...], o1_ref[...]], axis=1) + b1_ref[...]
    h1 = jnp.where(h1 > 0, h1, jnp.exp(jnp.minimum(h1, 0.0)) - 1.0)
    h2_ref[...] = jnp.dot(h1, w2_ref[...], preferred_element_type=jnp.float32)
    sd = jnp.dot(h1, v2_ref[...], preferred_element_type=jnp.float32)
    a2s = jnp.broadcast_to(sd[:, 0:1], (sd.shape[0], 16))
    a2d = jnp.broadcast_to(sd[:, 1:2], (sd.shape[0], 16))
    a2s_ref[...] = a2s
    a2d_ref[...] = a2d
    blkmax = jnp.concatenate(
        [jnp.max(a2s, axis=0)[None], jnp.max(a2d, axis=0)[None],
         jnp.full((6, 16), -3.0e38, jnp.float32)], axis=0)

    @pl.when(i == 0)
    def _():
        mx_ref[...] = jnp.full((8, 16), -3.0e38, jnp.float32)

    mx_ref[...] = jnp.maximum(mx_ref[...], blkmax)


def _k5(o0, o1, b1p, w2p, v2sd):
    f = pl.pallas_call(
        _k5_body,
        grid=(N // ROWBLK,),
        in_specs=[
            pl.BlockSpec((ROWBLK, 128), lambda i: (i, 0)),
            pl.BlockSpec((ROWBLK, 128), lambda i: (i, 0)),
            pl.BlockSpec((1, 256), lambda i: (0, 0)),
            pl.BlockSpec((256, 64), lambda i: (0, 0)),
            pl.BlockSpec((256, 2), lambda i: (0, 0)),
        ],
        out_specs=[
            pl.BlockSpec((ROWBLK, 64), lambda i: (i, 0)),
            pl.BlockSpec((ROWBLK, 16), lambda i: (i, 0)),
            pl.BlockSpec((ROWBLK, 16), lambda i: (i, 0)),
            pl.BlockSpec((8, 16), lambda i: (0, 0)),
        ],
        out_shape=[
            jax.ShapeDtypeStruct((N, 64), jnp.float32),
            jax.ShapeDtypeStruct((N, 16), jnp.float32),
            jax.ShapeDtypeStruct((N, 16), jnp.float32),
            jax.ShapeDtypeStruct((8, 16), jnp.float32),
        ],
    )
    return f(o0, o1, b1p, w2p, v2sd)


def _k7b_body(d0_ref, d1_ref, r_ref):
    r_ref[...] = 1.0 / (d0_ref[...] + d1_ref[...] + 1e-16)


def _k7b(d0, d1):
    f = pl.pallas_call(
        _k7b_body,
        grid=(NP // RPT,),
        in_specs=[
            pl.BlockSpec((RPT, 16), lambda i: (i, 0)),
            pl.BlockSpec((RPT, 16), lambda i: (i, 0)),
        ],
        out_specs=pl.BlockSpec((RPT, 16), lambda i: (i, 0)),
        out_shape=jax.ShapeDtypeStruct((NP, 16), jnp.float32),
    )
    return f(d0, d1)


def _k9_body(p0_ref, p1_ref, b2_ref, o_ref):
    z = p0_ref[...] + p1_ref[...] + b2_ref[...]
    m = jnp.max(z, axis=1, keepdims=True)
    ex = jnp.exp(z - m)
    s = jnp.sum(ex, axis=1, keepdims=True)
    o_ref[...] = z - m - jnp.log(s)


def _k9(p0, p1, b2):
    f = pl.pallas_call(
        _k9_body,
        grid=(N // ROWBLK,),
        in_specs=[
            pl.BlockSpec((ROWBLK, 64), lambda i: (i, 0)),
            pl.BlockSpec((ROWBLK, 64), lambda i: (i, 0)),
            pl.BlockSpec((1, 64), lambda i: (0, 0)),
        ],
        out_specs=pl.BlockSpec((ROWBLK, 64), lambda i: (i, 0)),
        out_shape=jax.ShapeDtypeStruct((N, 64), jnp.float32),
    )
    return f(p0, p1, b2)


def _zero16(buf_ref, rows):
    z = jnp.zeros((16,), jnp.float32)

    def row(i, _):
        buf_ref[i] = z
        return 0

    lax.fori_loop(0, rows, row, 0)


@functools.partial(
    pl.kernel,
    out_type=[
        jax.ShapeDtypeStruct((E, 16), jnp.float32),
        jax.ShapeDtypeStruct((E, 16), jnp.float32),
        jax.ShapeDtypeStruct((NP, 16), jnp.float32),
        jax.ShapeDtypeStruct((NP, 16), jnp.float32),
    ],
    mesh=_MESH,
    compiler_params=_SC_PARAMS,
    scratch_types=[
        pltpu.VMEM((CE,), jnp.int32),
        pltpu.VMEM((CE,), jnp.int32),
        pltpu.VMEM((CE, 16), jnp.float32),
        pltpu.VMEM((CE, 16), jnp.float32),
        pltpu.VMEM((CE, 16), jnp.float32),
        pltpu.VMEM((RPT, 16), jnp.float32),
        pltpu.VMEM((8, 16), jnp.float32),
        pltpu.VMEM_SHARED((NP, 16), jnp.float32),
        pltpu.SemaphoreType.DMA,
    ],
)
def _k2(as0, ad0, as1, ad1, mx1, src_r, dst_r, ex0, ex1, rec0, rec1,
        idxa, idxb, ga, gb, eb, zb, mxb, den_sh, sem):
    c = lax.axis_index("c")
    t = lax.axis_index("s")
    nch = EPT1 // CE

    pltpu.sync_copy(mx1, mxb)
    _zero16(zb, RPT)
    pltpu.sync_copy(zb, den_sh.at[pl.ds(t * RPT, RPT)])
    plsc.subcore_barrier()

    def half(s, as_ref, ad_ref, ex_ref, rec_ref):
        shift = jnp.maximum(mxb[2 * s] + mxb[2 * s + 1], 0.0)

        def chunk(j, _):
            base = t * EPT1 + j * CE
            pltpu.sync_copy(src_r.at[pl.ds(base, CE)], idxa)
            pltpu.async_copy(as_ref.at[idxa], ga, sem)
            pltpu.sync_copy(dst_r.at[pl.ds(base, CE)], idxb)
            pltpu.async_copy(ad_ref.at[idxb], gb, sem)
            pltpu.make_async_copy(as_ref.at[idxa], ga, sem).wait()
            pltpu.make_async_copy(ad_ref.at[idxb], gb, sem).wait()

            @plsc.parallel_loop(0, CE, unroll=4)
            def _(i):
                e = ga[i] + gb[i]
                e = jnp.where(e >= 0.0, e, 0.2 * e)
                eb[i] = jnp.exp(e - shift)

            pltpu.sync_copy(eb, ex_ref.at[pl.ds(base, CE)])
            pltpu.sync_copy(eb, den_sh.at[idxb], add=True)
            return 0

        lax.fori_loop(0, nch, chunk, 0)
        plsc.subcore_barrier()

        r0 = t * RPT
        pltpu.sync_copy(den_sh.at[pl.ds(r0, RPT)], zb)

        @plsc.parallel_loop(0, RPT, unroll=4)
        def _(i):
            zb[i] = 1.0 / (zb[i] + 1e-16)

        pltpu.sync_copy(zb, rec_ref.at[pl.ds(r0, RPT)])

    @pl.when(c == 0)
    def _():
        half(0, as0, ad0, ex0, rec0)

    @pl.when(c == 1)
    def _():
        half(1, as1, ad1, ex1, rec1)


@functools.partial(
    pl.kernel,
    out_type=[
        jax.ShapeDtypeStruct((NP, 128), jnp.float32),
        jax.ShapeDtypeStruct((NP, 128), jnp.float32),
    ],
    mesh=_MESH,
    compiler_params=_SC_PARAMS,
    scratch_types=[
        pltpu.VMEM((CM3,), jnp.int32),
        pltpu.VMEM((CM3,), jnp.int32),
        pltpu.VMEM((CM3,), jnp.int32),
        pltpu.VMEM((CM3,), jnp.int32),
        pltpu.VMEM((CM3, 16), jnp.float32),
        pltpu.VMEM((CM3, 16), jnp.float32),
        pltpu.VMEM((CM3, 16), jnp.float32),
        pltpu.VMEM((CM3, 16), jnp.float32),
        pltpu.VMEM((CM3, 128), jnp.float32),
        pltpu.VMEM((CM3, 128), jnp.float32),
        pltpu.VMEM((64, 128), jnp.float32),
        pltpu.VMEM_SHARED((NP, 128), jnp.float32),
        pltpu.SemaphoreType.DMA,
        pltpu.SemaphoreType.DMA,
        pltpu.SemaphoreType.DMA,
        pltpu.SemaphoreType.DMA,
    ],
)
def _k3(h0, h1f, ex0, ex1, rec0, rec1, src_r, dst_r, out0, out1,
        idxs0, idxd0, idxs1, idxd1, exb0, recb0, exb1, recb1,
        hb0, hb1, zb, osh, sem0, sem1, ssem0, ssem1):
    c = lax.axis_index("c")
    t = lax.axis_index("s")
    nch = EPT1 // CM3
    z = jnp.zeros((16,), jnp.float32)

    def zrow(i, _):
        for v in range(8):
            zb[i, pl.ds(v * 16, 16)] = z
        return 0

    lax.fori_loop(0, 64, zrow, 0)
    for k in range(RPT // 64):
        pltpu.sync_copy(zb, osh.at[pl.ds(t * RPT + k * 64, 64)])
    plsc.subcore_barrier()

    def half(h_ref, ex_ref, rec_ref, out_ref):
        bufs = ((idxs0, idxd0, hb0, recb0, exb0, sem0, ssem0),
                (idxs1, idxd1, hb1, recb1, exb1, sem1, ssem1))

        def prep(j, b):
            idxs, idxd, hb, recb, exb, sem, ssem = bufs[b]
            base = t * EPT1 + j * CM3
            pltpu.sync_copy(src_r.at[pl.ds(base, CM3)], idxs)
            pltpu.async_copy(h_ref.at[idxs], hb, sem)
            pltpu.sync_copy(dst_r.at[pl.ds(base, CM3)], idxd)
            pltpu.async_copy(rec_ref.at[idxd], recb, sem)
            pltpu.async_copy(ex_ref.at[pl.ds(base, CM3)], exb, sem)

        def wait_scat(b):
            idxs, idxd, hb, recb, exb, sem, ssem = bufs[b]
            pltpu.make_async_copy(hb, osh.at[idxd], ssem).wait()

        def consume(b):
            idxs, idxd, hb, recb, exb, sem, ssem = bufs[b]
            pltpu.make_async_copy(h_ref.at[idxs], hb, sem).wait()
            pltpu.make_async_copy(rec_ref.at[idxd], recb, sem).wait()
            pltpu.make_async_copy(ex_ref.at[pl.ds(0, CM3)], exb, sem).wait()

            @plsc.parallel_loop(0, CM3, unroll=4)
            def _(i):
                alpha = exb[i] * recb[i]
                for v in range(8):
                    sl = pl.ds(v * 16, 16)
                    hb[i, sl] = hb[i, sl] * alpha
            pltpu.async_copy(hb, osh.at[idxd], ssem, add=True)

        prep(0, 0)
        prep(1, 1)

        def pair(j2, _):
            consume(0)
            consume(1)

            @pl.when(j2 < nch // 2 - 1)
            def _():
                wait_scat(0)
                prep(2 * j2 + 2, 0)
                wait_scat(1)
                prep(2 * j2 + 3, 1)

            return 0

        lax.fori_loop(0, nch // 2, pair, 0)
        wait_scat(0)
        wait_scat(1)
        plsc.subcore_barrier()
        r0 = t * RPT
        pltpu.sync_copy(osh.at[pl.ds(r0, RPT)], out_ref.at[pl.ds(r0, RPT)])

    @pl.when(c == 0)
    def _():
        half(h0, ex0, rec0, out0)

    @pl.when(c == 1)
    def _():
        half(h1f, ex1, rec1, out1)


@functools.partial(
    pl.kernel,
    out_type=[
        jax.ShapeDtypeStruct((E, 16), jnp.float32),
        jax.ShapeDtypeStruct((NP, 16), jnp.float32),
        jax.ShapeDtypeStruct((NP, 16), jnp.float32),
    ],
    mesh=_MESH,
    compiler_params=_SC_PARAMS,
    scratch_types=[
        pltpu.VMEM((CE,), jnp.int32),
        pltpu.VMEM((CE,), jnp.int32),
        pltpu.VMEM((CE, 16), jnp.float32),
        pltpu.VMEM((CE, 16), jnp.float32),
        pltpu.VMEM((CE, 16), jnp.float32),
        pltpu.VMEM((RPT, 16), jnp.float32),
        pltpu.VMEM((8, 16), jnp.float32),
        pltpu.VMEM_SHARED((NP, 16), jnp.float32),
        pltpu.SemaphoreType.DMA,
    ],
)
def _k7(a2s, a2d, mx2, src_r, dst_r, ex2s, d20, d21,
        idxa, idxb, ga, gb, eb, zb, mxb, den_sh, sem):
    c = lax.axis_index("c")
    t = lax.axis_index("s")
    nch = EPT2 // CE

    pltpu.sync_copy(mx2, mxb)
    shift = jnp.maximum(mxb[0] + mxb[1], 0.0)

    _zero16(zb, RPT)
    pltpu.sync_copy(zb, den_sh.at[pl.ds(t * RPT, RPT)])
    plsc.subcore_barrier()

    def chunk(j, _):
        base = c * (E // 2) + t * EPT2 + j * CE
        pltpu.sync_copy(src_r.at[pl.ds(base, CE)], idxa)
        pltpu.async_copy(a2s.at[idxa], ga, sem).wait()
        pltpu.sync_copy(dst_r.at[pl.ds(base, CE)], idxb)
        pltpu.async_copy(a2d.at[idxb], gb, sem).wait()

        @plsc.parallel_loop(0, CE, unroll=4)
        def _(i):
            e = ga[i] + gb[i]
            e = jnp.where(e >= 0.0, e, 0.2 * e)
            eb[i] = jnp.exp(e - shift)
        pltpu.sync_copy(eb, ex2s.at[pl.ds(base, CE)])
        pltpu.sync_copy(eb, den_sh.at[idxb], add=True)
        return 0

    lax.fori_loop(0, nch, chunk, 0)
    plsc.subcore_barrier()

    r0 = t * RPT

    @pl.when(c == 0)
    def _():
        pltpu.sync_copy(den_sh.at[pl.ds(r0, RPT)], d20.at[pl.ds(r0, RPT)])

    @pl.when(c == 1)
    def _():
        pltpu.sync_copy(den_sh.at[pl.ds(r0, RPT)], d21.at[pl.ds(r0, RPT)])


@functools.partial(
    pl.kernel,
    out_type=[
        jax.ShapeDtypeStruct((NP, 64), jnp.float32),
        jax.ShapeDtypeStruct((NP, 64), jnp.float32),
    ],
    mesh=_MESH,
    compiler_params=_SC_PARAMS,
    scratch_types=[
        pltpu.VMEM((CM,), jnp.int32),
        pltpu.VMEM((CM,), jnp.int32),
        pltpu.VMEM((CM,), jnp.int32),
        pltpu.VMEM((CM,), jnp.int32),
        pltpu.VMEM((CM, 16), jnp.float32),
        pltpu.VMEM((CM, 16), jnp.float32),
        pltpu.VMEM((CM, 16), jnp.float32),
        pltpu.VMEM((CM, 16), jnp.float32),
        pltpu.VMEM((CM, 64), jnp.float32),
        pltpu.VMEM((CM, 64), jnp.float32),
        pltpu.VMEM((128, 64), jnp.float32),
        pltpu.VMEM_SHARED((NP, 64), jnp.float32),
        pltpu.SemaphoreType.DMA,
        pltpu.SemaphoreType.DMA,
        pltpu.SemaphoreType.DMA,
        pltpu.SemaphoreType.DMA,
    ],
)
def _k8(h2, ex2s, rec2, src_r, dst_r, o2p0, o2p1,
        idxs0, idxd0, idxs1, idxd1, exb0, recb0, exb1, recb1,
        hb0, hb1, zb, osh, sem0, sem1, ssem0, ssem1):
    c = lax.axis_index("c")
    t = lax.axis_index("s")
    nch = EPT2 // CM
    z = jnp.zeros((16,), jnp.float32)

    def zrow(i, _):
        for v in range(4):
            zb[i, pl.ds(v * 16, 16)] = z
        return 0

    lax.fori_loop(0, 128, zrow, 0)
    for k in range(RPT // 128):
        pltpu.sync_copy(zb, osh.at[pl.ds(t * RPT + k * 128, 128)])
    plsc.subcore_barrier()

    bufs = ((idxs0, idxd0, hb0, recb0, exb0, sem0, ssem0),
            (idxs1, idxd1, hb1, recb1, exb1, sem1, ssem1))

    def prep(j, b):
        idxs, idxd, hb, recb, exb, sem, ssem = bufs[b]
        base = c * (E // 2) + t * EPT2 + j * CM
        pltpu.sync_copy(src_r.at[pl.ds(base, CM)], idxs)
        pltpu.async_copy(h2.at[idxs], hb, sem)
        pltpu.sync_copy(dst_r.at[pl.ds(base, CM)], idxd)
        pltpu.async_copy(rec2.at[idxd], recb, sem)
        pltpu.async_copy(ex2s.at[pl.ds(base, CM)], exb, sem)

    def wait_scat(b):
        idxs, idxd, hb, recb, exb, sem, ssem = bufs[b]
        pltpu.make_async_copy(hb, osh.at[idxd], ssem).wait()

    def consume(b):
        idxs, idxd, hb, recb, exb, sem, ssem = bufs[b]
        pltpu.make_async_copy(h2.at[idxs], hb, sem).wait()
        pltpu.make_async_copy(rec2.at[idxd], recb, sem).wait()
        pltpu.make_async_copy(ex2s.at[pl.ds(0, CM)], exb, sem).wait()

        @plsc.parallel_loop(0, CM, unroll=4)
        def _(i):
            alpha = exb[i] * recb[i]
            for v in range(4):
                sl = pl.ds(v * 16, 16)
                hb[i, sl] = hb[i, sl] * alpha
        pltpu.async_copy(hb, osh.at[idxd], ssem, add=True)

    prep(0, 0)
    prep(1, 1)

    def pair(j2, _):
        consume(0)
        consume(1)

        @pl.when(j2 < nch // 2 - 1)
        def _():
            wait_scat(0)
            prep(2 * j2 + 2, 0)
            wait_scat(1)
            prep(2 * j2 + 3, 1)

        return 0

    lax.fori_loop(0, nch // 2, pair, 0)
    wait_scat(0)
    wait_scat(1)
    plsc.subcore_barrier()

    r0 = t * RPT

    @pl.when(c == 0)
    def _():
        pltpu.sync_copy(osh.at[pl.ds(r0, RPT)], o2p0.at[pl.ds(r0, RPT)])

    @pl.when(c == 1)
    def _():
        pltpu.sync_copy(osh.at[pl.ds(r0, RPT)], o2p1.at[pl.ds(r0, RPT)])


def kernel(x, edge_index, W1, att_src1, att_dst1, b1, W2, att_src2, att_dst2, b2):
    src = edge_index[0]
    dst = edge_index[1]

    perm = jnp.asarray(_PERM)
    w1p = W1[:, perm]
    v_src = jnp.sum(W1.reshape(128, H1, C1) * att_src1[None], axis=-1)
    v_dst = jnp.sum(W1.reshape(128, H1, C1) * att_dst1[None], axis=-1)
    rep = jnp.asarray(np.tile(np.arange(4), 4))
    wbig = jnp.concatenate(
        [w1p,
         v_src[:, 0 + rep], v_dst[:, 0 + rep],
         v_src[:, 4 + rep], v_dst[:, 4 + rep]], axis=1)
    b1p = b1[perm][None, :]
    w2p = W2[perm, :]
    v2sd = jnp.stack([w2p @ att_src2[0], w2p @ att_dst2[0]], axis=1)

    h0, h1f, as0, ad0, as1, ad1, mx1 = _k1(x, wbig)
    ex0, ex1, rec0, rec1 = _k2(as0, ad0, as1, ad1, mx1, src, dst)
    out0, out1 = _k3(h0, h1f, ex0, ex1, rec0, rec1, src, dst)

    h2, a2s_t, a2d_t, mx2 = _k5(out0, out1, b1p, w2p, v2sd)
    ex2s, d20, d21 = _k7(a2s_t, a2d_t, mx2, src, dst)
    rec2 = _k7b(d20, d21)
    o2p0, o2p1 = _k8(h2, ex2s, rec2, src, dst)
    return _k9(o2p0, o2p1, b2[None, :])


# speedup vs baseline: 58.8142x; 1.0269x over previous
import dataclasses
import functools

import jax
import jax.numpy as jnp
import numpy as np
from jax import lax
from jax.experimental import pallas as pl
from jax.experimental.pallas import tpu as pltpu
from jax.experimental.pallas import tpu_sc as plsc

N = 10000
E = 320000
NP = 10240
H1 = 8
C1 = 32
ROWBLK = 400

NSUB = 16
RPT = NP // NSUB

EPT1 = E // NSUB
EPT2 = E // (2 * NSUB)
CE1 = 160
CE2 = 400
ZR = 80

_MESH = plsc.VectorSubcoreMesh(core_axis_name="c", subcore_axis_name="s")

_SC_PARAMS = pltpu.CompilerParams()
if "needs_layout_passes" in pltpu.CompilerParams.__dataclass_fields__:
    _SC_PARAMS = dataclasses.replace(
        _SC_PARAMS, needs_layout_passes=False, use_tc_tiling_on_sc=False)


def _perm_half(s):
    p = np.arange(128)
    return (4 * s + (p % 4)) * C1 + (p // 4)


_PERM = np.concatenate([_perm_half(0), _perm_half(1)])


def _k1_body(x_ref, w_ref, h0_ref, h1_ref, as0_ref, ad0_ref, as1_ref, ad1_ref,
             mx_ref):
    i = pl.program_id(0)
    y = jnp.dot(x_ref[...], w_ref[...], preferred_element_type=jnp.float32)
    h0_ref[...] = y[:, 0:128]
    h1_ref[...] = y[:, 128:256]
    as0_ref[...] = y[:, 256:272]
    ad0_ref[...] = y[:, 272:288]
    as1_ref[...] = y[:, 288:304]
    ad1_ref[...] = y[:, 304:320]
    blkmax = jnp.concatenate(
        [jnp.max(y[:, 256 + 16 * k:272 + 16 * k], axis=0)[None] for k in range(4)]
        + [jnp.full((4, 16), -3.0e38, jnp.float32)], axis=0)

    @pl.when(i == 0)
    def _():
        mx_ref[...] = jnp.full((8, 16), -3.0e38, jnp.float32)

    mx_ref[...] = jnp.maximum(mx_ref[...], blkmax)


def _k1(x, wbig):
    f = pl.pallas_call(
        _k1_body,
        grid=(N // ROWBLK,),
        in_specs=[
            pl.BlockSpec((ROWBLK, 128), lambda i: (i, 0)),
            pl.BlockSpec((128, 320), lambda i: (0, 0)),
        ],
        out_specs=[
            pl.BlockSpec((ROWBLK, 128), lambda i: (i, 0)),
            pl.BlockSpec((ROWBLK, 128), lambda i: (i, 0)),
            pl.BlockSpec((ROWBLK, 16), lambda i: (i, 0)),
            pl.BlockSpec((ROWBLK, 16), lambda i: (i, 0)),
            pl.BlockSpec((ROWBLK, 16), lambda i: (i, 0)),
            pl.BlockSpec((ROWBLK, 16), lambda i: (i, 0)),
            pl.BlockSpec((8, 16), lambda i: (0, 0)),
        ],
        out_shape=[
            jax.ShapeDtypeStruct((N, 128), jnp.float32),
            jax.ShapeDtypeStruct((N, 128), jnp.float32),
            jax.ShapeDtypeStruct((N, 16), jnp.float32),
            jax.ShapeDtypeStruct((N, 16), jnp.float32),
            jax.ShapeDtypeStruct((N, 16), jnp.float32),
            jax.ShapeDtypeStruct((N, 16), jnp.float32),
            jax.ShapeDtypeStruct((8, 16), jnp.float32),
        ],
    )
    return f(x, wbig)


def _k5_body(o0_ref, o1_ref, r0_ref, r1_ref, b1_ref, w2_ref, v2_ref,
             h2_ref, a2s_ref, a2d_ref, mx_ref):
    i = pl.program_id(0)
    rec0 = jnp.concatenate([r0_ref[...]] * 8, axis=1)
    rec1 = jnp.concatenate([r1_ref[...]] * 8, axis=1)
    h1 = jnp.concatenate(
        [o0_ref[...] * rec0, o1_ref[...] * rec1], axis=1) + b1_ref[...]
    h1 = jnp.where(h1 > 0, h1, jnp.exp(jnp.minimum(h1, 0.0)) - 1.0)
    h2_ref[...] = jnp.dot(h1, w2_ref[...], preferred_element_type=jnp.float32)
    sd = jnp.dot(h1, v2_ref[...], preferred_element_type=jnp.float32)
    a2s = jnp.broadcast_to(sd[:, 0:1], (sd.shape[0], 16))
    a2d = jnp.broadcast_to(sd[:, 1:2], (sd.shape[0], 16))
    a2s_ref[...] = a2s
    a2d_ref[...] = a2d
    blkmax = jnp.concatenate(
        [jnp.max(a2s, axis=0)[None], jnp.max(a2d, axis=0)[None],
         jnp.full((6, 16), -3.0e38, jnp.float32)], axis=0)

    @pl.when(i == 0)
    def _():
        mx_ref[...] = jnp.full((8, 16), -3.0e38, jnp.float32)

    mx_ref[...] = jnp.maximum(mx_ref[...], blkmax)


def _k5(o0, o1, rec0, rec1, b1p, w2p, v2sd):
    f = pl.pallas_call(
        _k5_body,
        grid=(N // ROWBLK,),
        in_specs=[
            pl.BlockSpec((ROWBLK, 128), lambda i: (i, 0)),
            pl.BlockSpec((ROWBLK, 128), lambda i: (i, 0)),
            pl.BlockSpec((ROWBLK, 16), lambda i: (i, 0)),
            pl.BlockSpec((ROWBLK, 16), lambda i: (i, 0)),
            pl.BlockSpec((1, 256), lambda i: (0, 0)),
            pl.BlockSpec((256, 64), lambda i: (0, 0)),
            pl.BlockSpec((256, 2), lambda i: (0, 0)),
        ],
        out_specs=[
            pl.BlockSpec((ROWBLK, 64), lambda i: (i, 0)),
            pl.BlockSpec((ROWBLK, 16), lambda i: (i, 0)),
            pl.BlockSpec((ROWBLK, 16), lambda i: (i, 0)),
            pl.BlockSpec((8, 16), lambda i: (0, 0)),
        ],
        out_shape=[
            jax.ShapeDtypeStruct((N, 64), jnp.float32),
            jax.ShapeDtypeStruct((N, 16), jnp.float32),
            jax.ShapeDtypeStruct((N, 16), jnp.float32),
            jax.ShapeDtypeStruct((8, 16), jnp.float32),
        ],
    )
    return f(o0, o1, rec0, rec1, b1p, w2p, v2sd)


def _k9_body(p0_ref, p1_ref, d0_ref, d1_ref, b2_ref, o_ref):
    rec = 1.0 / (d0_ref[...] + d1_ref[...] + 1e-16)
    z = (p0_ref[...] + p1_ref[...]) * jnp.concatenate([rec] * 4, axis=1)
    z = z + b2_ref[...]
    m = jnp.max(z, axis=1, keepdims=True)
    ex = jnp.exp(z - m)
    s = jnp.sum(ex, axis=1, keepdims=True)
    o_ref[...] = z - m - jnp.log(s)


def _k9(p0, p1, d20, d21, b2):
    f = pl.pallas_call(
        _k9_body,
        grid=(N // ROWBLK,),
        in_specs=[
            pl.BlockSpec((ROWBLK, 64), lambda i: (i, 0)),
            pl.BlockSpec((ROWBLK, 64), lambda i: (i, 0)),
            pl.BlockSpec((ROWBLK, 16), lambda i: (i, 0)),
            pl.BlockSpec((ROWBLK, 16), lambda i: (i, 0)),
            pl.BlockSpec((1, 64), lambda i: (0, 0)),
        ],
        out_specs=pl.BlockSpec((ROWBLK, 64), lambda i: (i, 0)),
        out_shape=jax.ShapeDtypeStruct((N, 64), jnp.float32),
    )
    return f(p0, p1, d20, d21, b2)


@functools.partial(
    pl.kernel,
    out_type=[
        jax.ShapeDtypeStruct((NP, 128), jnp.float32),
        jax.ShapeDtypeStruct((NP, 128), jnp.float32),
        jax.ShapeDtypeStruct((NP, 16), jnp.float32),
        jax.ShapeDtypeStruct((NP, 16), jnp.float32),
    ],
    mesh=_MESH,
    compiler_params=_SC_PARAMS,
    scratch_types=[
        pltpu.VMEM((CE1,), jnp.int32),
        pltpu.VMEM((CE1,), jnp.int32),
        pltpu.VMEM((CE1, 16), jnp.float32),
        pltpu.VMEM((CE1, 16), jnp.float32),
        pltpu.VMEM((CE1, 16), jnp.float32),
        pltpu.VMEM((CE1, 128), jnp.float32),
        pltpu.VMEM((8, 16), jnp.float32),
        pltpu.VMEM_SHARED((NP, 16), jnp.float32),
        pltpu.VMEM_SHARED((NP, 128), jnp.float32),
        pltpu.SemaphoreType.DMA,
    ],
)
def _k23(as0, ad0, as1, ad1, mx1, h0, h1f, src_r, dst_r,
         out0, out1, rec0, rec1,
         idxa, idxb, ga, gb, eb, hb, mxb, den_sh, osh, sem):
    c = lax.axis_index("c")
    t = lax.axis_index("s")
    nch = EPT1 // CE1
    z = jnp.zeros((16,), jnp.float32)

    pltpu.sync_copy(mx1, mxb)

    def zrow(i, _):
        eb[i] = z
        for v in range(8):
            hb[i, pl.ds(v * 16, 16)] = z
        return 0

    lax.fori_loop(0, ZR, zrow, 0)
    for k in range(RPT // ZR):
        sl = pl.ds(t * RPT + k * ZR, ZR)
        pltpu.sync_copy(eb.at[pl.ds(0, ZR)], den_sh.at[sl])
        pltpu.sync_copy(hb.at[pl.ds(0, ZR)], osh.at[sl])
    plsc.subcore_barrier()

    def half(s, as_ref, ad_ref, h_ref, out_ref, rec_ref):
        shift = jnp.maximum(mxb[2 * s] + mxb[2 * s + 1], 0.0)

        def chunk(j, _):
            base = t * EPT1 + j * CE1
            pltpu.sync_copy(src_r.at[pl.ds(base, CE1)], idxa)
            pltpu.async_copy(as_ref.at[idxa], ga, sem)
            pltpu.async_copy(h_ref.at[idxa], hb, sem)
            pltpu.sync_copy(dst_r.at[pl.ds(base, CE1)], idxb)
            pltpu.async_copy(ad_ref.at[idxb], gb, sem)
            pltpu.make_async_copy(as_ref.at[idxa], ga, sem).wait()
            pltpu.make_async_copy(ad_ref.at[idxb], gb, sem).wait()
            pltpu.make_async_copy(h_ref.at[idxa], hb, sem).wait()

            @plsc.parallel_loop(0, CE1, unroll=4)
            def _(i):
                e = ga[i] + gb[i]
                e = jnp.where(e >= 0.0, e, 0.2 * e)
                ex = jnp.exp(e - shift)
                eb[i] = ex
                for v in range(8):
                    sl = pl.ds(v * 16, 16)
                    hb[i, sl] = hb[i, sl] * ex

            pltpu.sync_copy(eb, den_sh.at[idxb], add=True)
            pltpu.sync_copy(hb, osh.at[idxb], add=True)
            return 0

        lax.fori_loop(0, nch, chunk, 0)
        plsc.subcore_barrier()

        r0 = t * RPT
        pltpu.sync_copy(osh.at[pl.ds(r0, RPT)], out_ref.at[pl.ds(r0, RPT)])
        for k in range(RPT // ZR):
            sl = pl.ds(r0 + k * ZR, ZR)
            pltpu.sync_copy(den_sh.at[sl], eb.at[pl.ds(0, ZR)])

            @plsc.parallel_loop(0, ZR, unroll=4)
            def _(i):
                eb[i] = 1.0 / (eb[i] + 1e-16)

            pltpu.sync_copy(eb.at[pl.ds(0, ZR)], rec_ref.at[sl])

    @pl.when(c == 0)
    def _():
        half(0, as0, ad0, h0, out0, rec0)

    @pl.when(c == 1)
    def _():
        half(1, as1, ad1, h1f, out1, rec1)


@functools.partial(
    pl.kernel,
    out_type=[
        jax.ShapeDtypeStruct((NP, 64), jnp.float32),
        jax.ShapeDtypeStruct((NP, 64), jnp.float32),
        jax.ShapeDtypeStruct((NP, 16), jnp.float32),
        jax.ShapeDtypeStruct((NP, 16), jnp.float32),
    ],
    mesh=_MESH,
    compiler_params=_SC_PARAMS,
    scratch_types=[
        pltpu.VMEM((CE2,), jnp.int32),
        pltpu.VMEM((CE2,), jnp.int32),
        pltpu.VMEM((CE2, 16), jnp.float32),
        pltpu.VMEM((CE2, 16), jnp.float32),
        pltpu.VMEM((CE2, 16), jnp.float32),
        pltpu.VMEM((CE2, 64), jnp.float32),
        pltpu.VMEM((8, 16), jnp.float32),
        pltpu.VMEM_SHARED((NP, 16), jnp.float32),
        pltpu.VMEM_SHARED((NP, 64), jnp.float32),
        pltpu.SemaphoreType.DMA,
    ],
)
def _k78(a2s, a2d, mx2, h2, src_r, dst_r, o2p0, o2p1, d20, d21,
         idxa, idxb, ga, gb, eb, hb, mxb, den_sh, osh, sem):
    c = lax.axis_index("c")
    t = lax.axis_index("s")
    nch = EPT2 // CE2
    z = jnp.zeros((16,), jnp.float32)

    pltpu.sync_copy(mx2, mxb)
    shift = jnp.maximum(mxb[0] + mxb[1], 0.0)

    def zrow(i, _):
        eb[i] = z
        for v in range(4):
            hb[i, pl.ds(v * 16, 16)] = z
        return 0

    lax.fori_loop(0, ZR, zrow, 0)
    for k in range(RPT // ZR):
        sl = pl.ds(t * RPT + k * ZR, ZR)
        pltpu.sync_copy(eb.at[pl.ds(0, ZR)], den_sh.at[sl])
        pltpu.sync_copy(hb.at[pl.ds(0, ZR)], osh.at[sl])
    plsc.subcore_barrier()

    def chunk(j, _):
        base = c * (E // 2) + t * EPT2 + j * CE2
        pltpu.sync_copy(src_r.at[pl.ds(base, CE2)], idxa)
        pltpu.async_copy(a2s.at[idxa], ga, sem)
        pltpu.async_copy(h2.at[idxa], hb, sem)
        pltpu.sync_copy(dst_r.at[pl.ds(base, CE2)], idxb)
        pltpu.async_copy(a2d.at[idxb], gb, sem)
        pltpu.make_async_copy(a2s.at[idxa], ga, sem).wait()
        pltpu.make_async_copy(a2d.at[idxb], gb, sem).wait()
        pltpu.make_async_copy(h2.at[idxa], hb, sem).wait()

        @plsc.parallel_loop(0, CE2, unroll=4)
        def _(i):
            e = ga[i] + gb[i]
            e = jnp.where(e >= 0.0, e, 0.2 * e)
            ex = jnp.exp(e - shift)
            eb[i] = ex
            for v in range(4):
                sl = pl.ds(v * 16, 16)
                hb[i, sl] = hb[i, sl] * ex

        pltpu.sync_copy(eb, den_sh.at[idxb], add=True)
        pltpu.sync_copy(hb, osh.at[idxb], add=True)
        return 0

    lax.fori_loop(0, nch, chunk, 0)
    plsc.subcore_barrier()

    r0 = t * RPT

    @pl.when(c == 0)
    def _():
        pltpu.sync_copy(den_sh.at[pl.ds(r0, RPT)], d20.at[pl.ds(r0, RPT)])
        pltpu.sync_copy(osh.at[pl.ds(r0, RPT)], o2p0.at[pl.ds(r0, RPT)])

    @pl.when(c == 1)
    def _():
        pltpu.sync_copy(den_sh.at[pl.ds(r0, RPT)], d21.at[pl.ds(r0, RPT)])
        pltpu.sync_copy(osh.at[pl.ds(r0, RPT)], o2p1.at[pl.ds(r0, RPT)])


def kernel(x, edge_index, W1, att_src1, att_dst1, b1, W2, att_src2, att_dst2, b2):
    src = edge_index[0]
    dst = edge_index[1]

    perm = jnp.asarray(_PERM)
    w1p = W1[:, perm]
    v_src = jnp.sum(W1.reshape(128, H1, C1) * att_src1[None], axis=-1)
    v_dst = jnp.sum(W1.reshape(128, H1, C1) * att_dst1[None], axis=-1)
    rep = jnp.asarray(np.tile(np.arange(4), 4))
    wbig = jnp.concatenate(
        [w1p,
         v_src[:, 0 + rep], v_dst[:, 0 + rep],
         v_src[:, 4 + rep], v_dst[:, 4 + rep]], axis=1)
    b1p = b1[perm][None, :]
    w2p = W2[perm, :]
    v2sd = jnp.stack([w2p @ att_src2[0], w2p @ att_dst2[0]], axis=1)

    h0, h1f, as0, ad0, as1, ad1, mx1 = _k1(x, wbig)
    out0, out1, rec0, rec1 = _k23(as0, ad0, as1, ad1, mx1, h0, h1f, src, dst)

    h2, a2s_t, a2d_t, mx2 = _k5(out0, out1, rec0, rec1, b1p, w2p, v2sd)
    o2p0, o2p1, d20, d21 = _k78(a2s_t, a2d_t, mx2, h2, src, dst)
    return _k9(o2p0, o2p1, d20, d21, b2[None, :])
